# baseline (speedup 1.0000x reference)
import numpy as np
import jax
import jax.numpy as jnp
from jax import lax
from jax.experimental import pallas as pl
from jax.experimental.pallas import tpu as pltpu
from jax.experimental.pallas import tpu_sc as plsc

_N = 10000
_NG = 64
_NPAD = 10112
_E = 320000
_EPAD = 327680
_C = 128
_CF = 64
_RPT = _NPAD // 16
_NCH = _EPAD // (16 * _CF)
_NPAIR = _NCH // 2
_NGRP = _NCH // 8
_INV = 1.0 / np.sqrt(32.0)

_MESH = plsc.VectorSubcoreMesh(
    core_axis_name="c", subcore_axis_name="s", num_cores=2, num_subcores=16)
_SC_PARAMS = pltpu.CompilerParams(use_tc_tiling_on_sc=False,
                                  needs_layout_passes=False)

_f32 = jnp.float32


def _silu(v):
    return v * lax.logistic(v)


def _split4(v):
    bf = jnp.bfloat16
    return (v[:, :128].astype(bf), v[:, 128:144],
            v[:, 160:288].astype(bf), v[:, 144:160])


def _cat4(m0, t0, m1, t1):
    return jnp.concatenate([m0, t0, t1, m1], axis=1)


_QN = np.arange(288)
for _base in (0, 160):
    for _j in range(4):
        _blk = _base + 32 * _j
        _QN[_blk:_blk + 16] = _blk + 2 * np.arange(16)
        _QN[_blk + 16:_blk + 32] = _blk + 2 * np.arange(16) + 1


def _radial_call(emb, W0, W1, W2, W3, split):
    BE = 1024
    d1 = W3.shape[1]

    def body(emb_ref, w0_ref, w1_ref, w2_ref, w3_ref, *outs):
        bf = jnp.bfloat16
        v = _silu(jnp.dot(emb_ref[...], w0_ref[...], preferred_element_type=_f32))
        v = _silu(jnp.dot(v.astype(bf), w1_ref[...].astype(bf),
                          preferred_element_type=_f32))
        v = _silu(jnp.dot(v.astype(bf), w2_ref[...].astype(bf),
                          preferred_element_type=_f32))
        v = jnp.dot(v.astype(bf), w3_ref[...].astype(bf),
                    preferred_element_type=_f32)
        if split:
            m0, t0, m1, t1 = _split4(v)
            outs[0][...], outs[1][...], outs[2][...], outs[3][...] = m0, t0, m1, t1
        else:
            outs[0][...] = v

    def full(shp):
        return pl.BlockSpec(shp, lambda i: (0, 0))

    in_specs = [pl.BlockSpec((BE, emb.shape[1]), lambda i: (i, 0)),
                full(W0.shape), full(W1.shape), full(W2.shape), full(W3.shape)]
    if split:
        out_shape = [jax.ShapeDtypeStruct((_EPAD, 128), jnp.bfloat16),
                     jax.ShapeDtypeStruct((_EPAD, 16), _f32),
                     jax.ShapeDtypeStruct((_EPAD, 128), jnp.bfloat16),
                     jax.ShapeDtypeStruct((_EPAD, 16), _f32)]
        out_specs = [pl.BlockSpec((BE, 128), lambda i: (i, 0)),
                     pl.BlockSpec((BE, 16), lambda i: (i, 0)),
                     pl.BlockSpec((BE, 128), lambda i: (i, 0)),
                     pl.BlockSpec((BE, 16), lambda i: (i, 0))]
    else:
        out_shape = [jax.ShapeDtypeStruct((_EPAD, d1), _f32)]
        out_specs = [pl.BlockSpec((BE, d1), lambda i: (i, 0))]
    return pl.pallas_call(
        body, grid=(_EPAD // BE,), in_specs=in_specs, out_specs=out_specs,
        out_shape=out_shape)(emb, W0, W1, W2, W3)


def _msg_split_call(h, W):
    BN = 128

    def body(h_ref, w_ref, o0, o1, o2, o3):
        hm = jnp.dot(h_ref[...], w_ref[...], preferred_element_type=_f32)
        m0, t0, m1, t1 = _split4(hm)
        o0[...], o1[...], o2[...], o3[...] = m0, t0, m1, t1

    return pl.pallas_call(
        body, grid=(_NPAD // BN,),
        in_specs=[pl.BlockSpec((BN, h.shape[1]), lambda i: (i, 0)),
                  pl.BlockSpec(W.shape, lambda i: (0, 0))],
        out_specs=[pl.BlockSpec((BN, 128), lambda i: (i, 0)),
                   pl.BlockSpec((BN, 16), lambda i: (i, 0)),
                   pl.BlockSpec((BN, 128), lambda i: (i, 0)),
                   pl.BlockSpec((BN, 16), lambda i: (i, 0))],
        out_shape=[jax.ShapeDtypeStruct((_NPAD, 128), jnp.bfloat16),
                   jax.ShapeDtypeStruct((_NPAD, 16), _f32),
                   jax.ShapeDtypeStruct((_NPAD, 128), jnp.bfloat16),
                   jax.ShapeDtypeStruct((_NPAD, 16), _f32)])(h, W)


def _node_call(h, aggs, ea0, ea1, na, Wself, WedgeP, Wattr, Wmsg_next,
               *, cat, do_silu, split_next):
    BN = 128
    d1 = Wself.shape[1]
    nagg = len(aggs)

    def body(h_ref, *rest):
        agg_refs = rest[:nagg]
        e0_ref, e1_ref, na_ref, ws_ref, we_ref, wa_ref = rest[nagg:nagg + 6]
        rest = rest[nagg + 6:]
        if Wmsg_next is not None:
            wm_ref, outs = rest[0], rest[1:]
        else:
            outs = rest
        if cat:
            agg = _cat4(*(r[...] for r in agg_refs))
        else:
            agg = agg_refs[0][...] + agg_refs[1][...]
        ea = e0_ref[...] + e1_ref[...]
        z = jnp.dot(h_ref[...], ws_ref[...], preferred_element_type=_f32)
        z = z + (agg + jnp.dot(ea, we_ref[...], preferred_element_type=_f32)) * _INV
        z = z + na_ref[...][:, :1] * wa_ref[...]
        if do_silu:
            z = _silu(z)
        outs[0][...] = z
        if Wmsg_next is not None:
            hm = jnp.dot(z, wm_ref[...], preferred_element_type=_f32)
            if split_next:
                m0, t0, m1, t1 = _split4(hm)
                outs[1][...], outs[2][...], outs[3][...], outs[4][...] = \
                    m0, t0, m1, t1
            else:
                outs[1][...] = hm

    def full(shp):
        return pl.BlockSpec(shp, lambda i: (0, 0))

    def rowblk(w):
        return pl.BlockSpec((BN, w), lambda i: (i, 0))

    in_specs = [rowblk(h.shape[1])]
    in_specs += [rowblk(a.shape[1]) for a in aggs]
    in_specs += [rowblk(16), rowblk(16), rowblk(16),
                 full(Wself.shape), full(WedgeP.shape), full(Wattr.shape)]
    args = [h, *aggs, ea0, ea1, na, Wself, WedgeP, Wattr]
    out_shape = [jax.ShapeDtypeStruct((_NPAD, d1), _f32)]
    out_specs = [rowblk(d1)]
    if Wmsg_next is not None:
        in_specs.append(full(Wmsg_next.shape))
        args.append(Wmsg_next)
        if split_next:
            out_shape += [jax.ShapeDtypeStruct((_NPAD, 128), jnp.bfloat16),
                          jax.ShapeDtypeStruct((_NPAD, 16), _f32),
                          jax.ShapeDtypeStruct((_NPAD, 128), jnp.bfloat16),
                          jax.ShapeDtypeStruct((_NPAD, 16), _f32)]
            out_specs += [rowblk(128), rowblk(16), rowblk(128), rowblk(16)]
        else:
            dn = Wmsg_next.shape[1]
            out_shape.append(jax.ShapeDtypeStruct((_NPAD, dn), _f32))
            out_specs.append(rowblk(dn))
    return pl.pallas_call(
        body, grid=(_NPAD // BN,), in_specs=in_specs, out_specs=out_specs,
        out_shape=out_shape)(*args)


def _softmax_call(pool):
    def body(p_ref, o_ref):
        p = p_ref[0] + p_ref[1]
        s = p[:_NG, :10]
        m = jnp.max(s, axis=1, keepdims=True)
        e = jnp.exp(s - m)
        o_ref[...] = e / jnp.sum(e, axis=1, keepdims=True)

    return pl.pallas_call(
        body, out_shape=jax.ShapeDtypeStruct((_NG, 10), _f32))(pool)


def _zero_buf(buf, ncols):
    z16 = jnp.zeros((16,), _f32)

    def zrow(i, cc):
        for j in range(ncols // 16):
            buf[i, pl.ds(j * 16, 16)] = z16
        return cc
    lax.fori_loop(0, buf.shape[0], zrow, 0)


def _tiled_copy(src_getter, dst_getter, nrows):
    full_copies = _RPT // nrows
    rem = _RPT - full_copies * nrows
    for t in range(full_copies):
        pltpu.sync_copy(src_getter(t * nrows, nrows), dst_getter(t * nrows, nrows))
    if rem:
        off = full_copies * nrows
        pltpu.sync_copy(src_getter(off, rem), dst_getter(off, rem))


def _edge_fs_call(hms, ws, src2, dst2):
    def body(hm0_ref, ht0_ref, hm1_ref, ht1_ref,
             wm0_ref, wt0_ref, wm1_ref, wt1_ref, src_ref, dst_ref,
             om0_ref, ot0_ref, om1_ref, ot1_ref,
             sidx, didx, rowm, rowt, wbm, wbt, prodm, accm, acct,
             semg0, semg1, semw0, semw1, sems):
        c = lax.axis_index("c")
        s = lax.axis_index("s")
        tile_row = s * _NCH
        tile_edge = s * _NCH * _CF

        _zero_buf(prodm.at[0], 128)
        _zero_buf(rowt.at[0], 16)
        _tiled_copy(lambda o, n: prodm.at[0].at[pl.ds(0, n)],
                    lambda o, n: accm.at[pl.ds(s * _RPT + o, n)], _CF)
        _tiled_copy(lambda o, n: rowt.at[0].at[pl.ds(0, n)],
                    lambda o, n: acct.at[pl.ds(s * _RPT + o, n)], _CF)
        plsc.subcore_barrier()

        semg = (semg0, semg1)
        semw = (semw0, semw1)

        def main(hm_ref, ht_ref, wm_ref, wt_ref, om_ref, ot_ref):
            def g_start(buf, ib, islot):
                pltpu.async_copy(hm_ref.at[sidx.at[ib, islot]], rowm.at[buf],
                                 semg[buf])
                pltpu.async_copy(ht_ref.at[sidx.at[ib, islot]], rowt.at[buf],
                                 semg[buf])

            def g_wait(buf):
                pltpu.make_async_copy(
                    hm_ref.at[sidx.at[0, 0]], rowm.at[buf], semg[buf]).wait()
                pltpu.make_async_copy(
                    ht_ref.at[sidx.at[0, 0]], rowt.at[buf], semg[buf]).wait()

            def w_start(ebase, buf):
                pltpu.async_copy(wm_ref.at[pl.ds(ebase, _CF)], wbm.at[buf],
                                 semw[buf])
                pltpu.async_copy(wt_ref.at[pl.ds(ebase, _CF)], wbt.at[buf],
                                 semw[buf])

            def w_wait(buf):
                pltpu.make_async_copy(
                    wm_ref.at[pl.ds(tile_edge, _CF)], wbm.at[buf], semw[buf]).wait()
                pltpu.make_async_copy(
                    wt_ref.at[pl.ds(tile_edge, _CF)], wbt.at[buf], semw[buf]).wait()

            def s_start(buf, ib, islot):
                pltpu.async_copy(prodm.at[buf], accm.at[didx.at[ib, islot]],
                                 sems, add=True)
                pltpu.async_copy(rowt.at[buf], acct.at[didx.at[ib, islot]],
                                 sems, add=True)

            def s_wait():
                pltpu.make_async_copy(
                    prodm.at[0], accm.at[didx.at[0, 0]], sems).wait()
                pltpu.make_async_copy(
                    rowt.at[0], acct.at[didx.at[0, 0]], sems).wait()

            def mul(buf):
                def mrow(i, c2):
                    for j in range(4):
                        hv = rowm[buf, i, pl.ds(32 * j, 32)]
                        wv = wbm[buf, i, pl.ds(32 * j, 32)]
                        ha, hb = plsc.unpack(
                            hv, format=plsc.PackFormat.INTERLEAVED)
                        wa, wb = plsc.unpack(
                            wv, format=plsc.PackFormat.INTERLEAVED)
                        prodm[buf, i, pl.ds(32 * j, 16)] = ha * wa
                        prodm[buf, i, pl.ds(32 * j + 16, 16)] = hb * wb
                    sl = pl.ds(0, 16)
                    rowt[buf, i, sl] = rowt[buf, i, sl] * wbt[buf, i, sl]
                    return c2
                lax.fori_loop(0, _CF, mrow, 0)

            pltpu.sync_copy(src_ref.at[pl.ds(tile_row, 8)], sidx.at[0])
            pltpu.sync_copy(dst_ref.at[pl.ds(tile_row, 8)], didx.at[0])
            g_start(0, 0, 0)
            w_start(tile_edge, 0)

            def pair(p, cc):
                a = 2 * p
                b = a + 1
                q = lax.rem(p, 4)
                grp = lax.div(p, 4)

                @pl.when(jnp.logical_and(q == 0, grp + 1 < _NGRP))
                def _():
                    row = tile_row + (grp + 1) * 8
                    pltpu.sync_copy(src_ref.at[pl.ds(row, 8)],
                                    sidx.at[lax.rem(grp + 1, 2)])
                    pltpu.sync_copy(dst_ref.at[pl.ds(row, 8)],
                                    didx.at[lax.rem(grp + 1, 2)])

                ib = lax.rem(grp, 2)

                @pl.when(p > 0)
                def _():
                    s_wait()
                g_start(1, ib, lax.rem(b, 8))
                w_start(tile_edge + b * _CF, 1)
                g_wait(0)
                w_wait(0)
                mul(0)
                s_start(0, ib, lax.rem(a, 8))

                s_wait()
                @pl.when(p + 1 < _NPAIR)
                def _():
                    nk = a + 2
                    g_start(0, lax.rem(lax.div(nk, 8), 2), lax.rem(nk, 8))
                    w_start(tile_edge + nk * _CF, 0)
                g_wait(1)
                w_wait(1)
                mul(1)
                s_start(1, ib, lax.rem(b, 8))
                return cc

            lax.fori_loop(0, _NPAIR, pair, 0)
            s_wait()
            plsc.subcore_barrier()
            _tiled_copy(lambda o, n: accm.at[pl.ds(s * _RPT + o, n)],
                        lambda o, n: om_ref.at[pl.ds(s * _RPT + o, n)], 128)
            _tiled_copy(lambda o, n: acct.at[pl.ds(s * _RPT + o, n)],
                        lambda o, n: ot_ref.at[pl.ds(s * _RPT + o, n)], 128)

        @pl.when(c == 0)
        def _():
            main(hm0_ref, ht0_ref, wm0_ref, wt0_ref, om0_ref, ot0_ref)

        @pl.when(c == 1)
        def _():
            main(hm1_ref, ht1_ref, wm1_ref, wt1_ref, om1_ref, ot1_ref)

    return pl.kernel(
        body,
        out_type=[jax.ShapeDtypeStruct((_NPAD, 128), _f32),
                  jax.ShapeDtypeStruct((_NPAD, 16), _f32),
                  jax.ShapeDtypeStruct((_NPAD, 128), _f32),
                  jax.ShapeDtypeStruct((_NPAD, 16), _f32)],
        mesh=_MESH,
        compiler_params=_SC_PARAMS,
        scratch_types=[
            pltpu.VMEM((2, 8, _CF), jnp.int32),
            pltpu.VMEM((2, 8, _CF), jnp.int32),
            pltpu.VMEM((2, _CF, 128), jnp.bfloat16),
            pltpu.VMEM((2, _CF, 16), _f32),
            pltpu.VMEM((2, _CF, 128), jnp.bfloat16),
            pltpu.VMEM((2, _CF, 16), _f32),
            pltpu.VMEM((2, _CF, 128), _f32),
            pltpu.VMEM_SHARED((_NPAD, 128), _f32),
            pltpu.VMEM_SHARED((_NPAD, 16), _f32),
            pltpu.SemaphoreType.DMA,
            pltpu.SemaphoreType.DMA,
            pltpu.SemaphoreType.DMA,
            pltpu.SemaphoreType.DMA,
            pltpu.SemaphoreType.DMA,
        ])(*hms, *ws, src2, dst2)


def _edge_es_call(hm4, w4, srcp, dstp):
    NCH = _EPAD // (32 * _C)

    def body(hm_ref, w_ref, src_ref, dst_ref, out_ref,
             sidx, didx, rows, wbuf, acc, sem):
        c = lax.axis_index("c")
        s = lax.axis_index("s")
        wid = c * 16 + s

        _zero_buf(rows, 16)
        _tiled_copy(lambda o, n: rows.at[pl.ds(0, n)],
                    lambda o, n: acc.at[pl.ds(s * _RPT + o, n)], _C)
        plsc.subcore_barrier()

        def chunk(k, cc):
            base = (wid * NCH + k) * _C
            pltpu.sync_copy(src_ref.at[pl.ds(base, _C)], sidx)
            pltpu.sync_copy(dst_ref.at[pl.ds(base, _C)], didx)
            cp = pltpu.async_copy(hm_ref.at[sidx], rows, sem)
            pltpu.sync_copy(w_ref.at[pl.ds(base, _C)], wbuf)
            cp.wait()

            def mrow(i, c2):
                sl = pl.ds(0, 16)
                rows[i, sl] = rows[i, sl] * wbuf[i, sl]
                return c2
            lax.fori_loop(0, _C, mrow, 0)
            pltpu.sync_copy(rows, acc.at[didx], add=True)
            return cc
        lax.fori_loop(0, NCH, chunk, 0)
        plsc.subcore_barrier()
        _tiled_copy(lambda o, n: acc.at[pl.ds(s * _RPT + o, n)],
                    lambda o, n: out_ref.at[c, pl.ds(s * _RPT + o, n)], 128)

    return pl.kernel(
        body,
        out_type=jax.ShapeDtypeStruct((2, _NPAD, 16), _f32),
        mesh=_MESH,
        compiler_params=_SC_PARAMS,
        scratch_types=[
            pltpu.VMEM((_C,), jnp.int32),
            pltpu.VMEM((_C,), jnp.int32),
            pltpu.VMEM((_C, 16), _f32),
            pltpu.VMEM((_C, 16), _f32),
            pltpu.VMEM_SHARED((_NPAD, 16), _f32),
            pltpu.SemaphoreType.DMA,
        ])(hm4, w4, srcp, dstp)


def _ea_call(eap, dstp):
    NCH = _EPAD // (32 * _C)

    def body(ea_ref, dst_ref, out_ref, didx, rows, acc, sem):
        c = lax.axis_index("c")
        s = lax.axis_index("s")
        wid = c * 16 + s

        _zero_buf(rows, 16)
        _tiled_copy(lambda o, n: rows.at[pl.ds(0, n)],
                    lambda o, n: acc.at[pl.ds(s * _RPT + o, n)], _C)
        plsc.subcore_barrier()

        def chunk(k, cc):
            base = (wid * NCH + k) * _C
            pltpu.sync_copy(dst_ref.at[pl.ds(base, _C)], didx)
            pltpu.sync_copy(ea_ref.at[pl.ds(base, _C)], rows)
            pltpu.sync_copy(rows, acc.at[didx], add=True)
            return cc
        lax.fori_loop(0, NCH, chunk, 0)
        plsc.subcore_barrier()
        _tiled_copy(lambda o, n: acc.at[pl.ds(s * _RPT + o, n)],
                    lambda o, n: out_ref.at[c, pl.ds(s * _RPT + o, n)], 128)

    return pl.kernel(
        body,
        out_type=jax.ShapeDtypeStruct((2, _NPAD, 16), _f32),
        mesh=_MESH,
        compiler_params=_SC_PARAMS,
        scratch_types=[
            pltpu.VMEM((_C,), jnp.int32),
            pltpu.VMEM((_C, 16), _f32),
            pltpu.VMEM_SHARED((_NPAD, 16), _f32),
            pltpu.SemaphoreType.DMA,
        ])(eap, dstp)


def _pool_call(h4, batchp):
    CP = 64
    NCHT = _NPAD // CP

    def body(h_ref, b_ref, out_ref, bidx, rows, zbuf, acc, sem):
        c = lax.axis_index("c")
        s = lax.axis_index("s")
        wid = c * 16 + s

        @pl.when(s == 0)
        def _():
            _zero_buf(zbuf, 16)
            pltpu.sync_copy(zbuf, acc)
        plsc.subcore_barrier()

        def chunk(k, cc):
            idx = k * 32 + wid

            @pl.when(idx < NCHT)
            def _():
                base = idx * CP
                pltpu.sync_copy(b_ref.at[pl.ds(base, CP)], bidx)
                pltpu.sync_copy(h_ref.at[pl.ds(base, CP)], rows)
                pltpu.sync_copy(rows, acc.at[bidx], add=True)
            return cc
        lax.fori_loop(0, (NCHT + 31) // 32, chunk, 0)
        plsc.subcore_barrier()

        @pl.when(s == 0)
        def _():
            pltpu.sync_copy(acc, out_ref.at[c])

    return pl.kernel(
        body,
        out_type=jax.ShapeDtypeStruct((2, 72, 16), _f32),
        mesh=_MESH,
        compiler_params=_SC_PARAMS,
        scratch_types=[
            pltpu.VMEM((CP,), jnp.int32),
            pltpu.VMEM((CP, 16), _f32),
            pltpu.VMEM((72, 16), _f32),
            pltpu.VMEM_SHARED((72, 16), _f32),
            pltpu.SemaphoreType.DMA,
        ])(h4, batchp)


def kernel(x, node_attr, edge_src, edge_dst, edge_attr, edge_length_embedding,
           batch, params):
    xp = jnp.zeros((_NPAD, 128), _f32).at[:_N].set(x)
    nap = jnp.zeros((_NPAD, 16), _f32).at[:_N].set(
        jnp.broadcast_to(node_attr, (_N, 16)))
    srcp = jnp.full((_EPAD,), _N, jnp.int32).at[:_E].set(edge_src.astype(jnp.int32))
    dstp = jnp.full((_EPAD,), _N, jnp.int32).at[:_E].set(edge_dst.astype(jnp.int32))
    src2 = srcp.reshape(_EPAD // _CF, _CF)
    dst2 = dstp.reshape(_EPAD // _CF, _CF)
    eap = jnp.zeros((_EPAD, 16), _f32).at[:_E, :9].set(edge_attr)
    embp = jnp.zeros((_EPAD, 10), _f32).at[:_E].set(edge_length_embedding)
    batchp = jnp.full((_NPAD,), _NG, jnp.int32).at[:_N].set(batch.astype(jnp.int32))

    Q = jnp.asarray(_QN)

    def wedgeP(p):
        W = jnp.zeros((16, p['W_edge'].shape[1]), _f32).at[:9].set(p['W_edge'])
        return W[:, Q]

    p4 = params[3]
    Wself4 = jnp.zeros((288, 16), _f32).at[:, :10].set(p4['W_self'])[Q]
    Wedge4 = jnp.zeros((16, 16), _f32).at[:9, :10].set(p4['W_edge'])
    Wattr4 = jnp.zeros((1, 16), _f32).at[:, :10].set(p4['W_attr'])
    fcW3_4 = jnp.zeros((128, 16), _f32).at[:, :10].set(p4['fc_W3'])
    Wmsg4 = jnp.zeros((288, 16), _f32).at[:, :10].set(p4['W_msg'])[Q]

    ea_pair = _ea_call(eap, dstp)
    ea0, ea1 = ea_pair[0], ea_pair[1]

    hms = _msg_split_call(xp, params[0]['W_msg'])
    h = xp
    for li in range(3):
        p = params[li]
        ws = _radial_call(embp, p['fc_W0'], p['fc_W1'], p['fc_W2'],
                          p['fc_W3'], split=True)
        aggs = _edge_fs_call(hms, ws, src2, dst2)
        Wself_p = p['W_self'][:, Q] if li == 0 else p['W_self'][Q][:, Q]
        if li < 2:
            h, *hms = _node_call(
                h, aggs, ea0, ea1, nap, Wself_p, wedgeP(p),
                p['W_attr'][:, Q], params[li + 1]['W_msg'][Q],
                cat=True, do_silu=True, split_next=True)
        else:
            h, hm4 = _node_call(
                h, aggs, ea0, ea1, nap, Wself_p, wedgeP(p),
                p['W_attr'][:, Q], Wmsg4, cat=True, do_silu=True,
                split_next=False)

    (w4,) = _radial_call(embp, p4['fc_W0'], p4['fc_W1'], p4['fc_W2'], fcW3_4,
                         split=False)
    agg4 = _edge_es_call(hm4, w4, srcp, dstp)
    (h4,) = _node_call(h, [agg4[0], agg4[1]], ea0, ea1, nap, Wself4, Wedge4,
                       Wattr4, None, cat=False, do_silu=False, split_next=False)
    pool = _pool_call(h4, batchp)
    return _softmax_call(pool)


# speedup vs baseline: 1.0985x; 1.0985x over previous
import numpy as np
import jax
import jax.numpy as jnp
from jax import lax
from jax.experimental import pallas as pl
from jax.experimental.pallas import tpu as pltpu
from jax.experimental.pallas import tpu_sc as plsc

_N = 10000
_NG = 64
_NPAD = 10112
_E = 320000
_EPAD = 327680
_C = 128
_CF = 64
_RPT = _NPAD // 16
_NCH = _EPAD // (16 * _CF)
_NPAIR = _NCH // 2
_NGRP = _NCH // 8
_INV = 1.0 / np.sqrt(32.0)

_MESH = plsc.VectorSubcoreMesh(
    core_axis_name="c", subcore_axis_name="s", num_cores=2, num_subcores=16)
_SC_PARAMS = pltpu.CompilerParams(use_tc_tiling_on_sc=False,
                                  needs_layout_passes=False)

_f32 = jnp.float32


def _silu(v):
    return v * lax.logistic(v)


def _split4(v):
    return v[:, :128], v[:, 128:144], v[:, 160:288], v[:, 144:160]


def _cat4(m0, t0, m1, t1):
    return jnp.concatenate([m0, t0, t1, m1], axis=1)


def _radial_call(embT, W0, W1, W2, W3, split):
    BE = 1024
    d1 = W3.shape[1]

    def body(emb_ref, w0_ref, w1_ref, w2_ref, w3_ref, *outs):
        bf = jnp.bfloat16
        v = _silu(lax.dot_general(
            emb_ref[...], w0_ref[...], (((0,), (0,)), ((), ())),
            preferred_element_type=_f32))
        v = _silu(jnp.dot(v.astype(bf), w1_ref[...].astype(bf),
                          preferred_element_type=_f32))
        v = _silu(jnp.dot(v.astype(bf), w2_ref[...].astype(bf),
                          preferred_element_type=_f32))
        v = jnp.dot(v.astype(bf), w3_ref[...].astype(bf),
                    preferred_element_type=_f32)
        if split:
            m0, t0, m1, t1 = _split4(v)
            outs[0][...], outs[1][...], outs[2][...], outs[3][...] = m0, t0, m1, t1
        else:
            outs[0][...] = v

    def full(shp):
        return pl.BlockSpec(shp, lambda i: (0, 0))

    in_specs = [pl.BlockSpec((embT.shape[0], BE), lambda i: (0, i)),
                full(W0.shape), full(W1.shape), full(W2.shape), full(W3.shape)]
    if split:
        out_shape = [jax.ShapeDtypeStruct((_EPAD, 128), _f32),
                     jax.ShapeDtypeStruct((_EPAD, 16), _f32),
                     jax.ShapeDtypeStruct((_EPAD, 128), _f32),
                     jax.ShapeDtypeStruct((_EPAD, 16), _f32)]
        out_specs = [pl.BlockSpec((BE, 128), lambda i: (i, 0)),
                     pl.BlockSpec((BE, 16), lambda i: (i, 0)),
                     pl.BlockSpec((BE, 128), lambda i: (i, 0)),
                     pl.BlockSpec((BE, 16), lambda i: (i, 0))]
    else:
        out_shape = [jax.ShapeDtypeStruct((_EPAD, d1), _f32)]
        out_specs = [pl.BlockSpec((BE, d1), lambda i: (i, 0))]
    return pl.pallas_call(
        body, grid=(_EPAD // BE,), in_specs=in_specs, out_specs=out_specs,
        out_shape=out_shape)(embT, W0, W1, W2, W3)


def _msg_split_call(h, W):
    BN = 128

    def body(h_ref, w_ref, o0, o1, o2, o3):
        hm = jnp.dot(h_ref[...], w_ref[...], preferred_element_type=_f32)
        m0, t0, m1, t1 = _split4(hm)
        o0[...], o1[...], o2[...], o3[...] = m0, t0, m1, t1

    return pl.pallas_call(
        body, grid=(_NPAD // BN,),
        in_specs=[pl.BlockSpec((BN, h.shape[1]), lambda i: (i, 0)),
                  pl.BlockSpec(W.shape, lambda i: (0, 0))],
        out_specs=[pl.BlockSpec((BN, 128), lambda i: (i, 0)),
                   pl.BlockSpec((BN, 16), lambda i: (i, 0)),
                   pl.BlockSpec((BN, 128), lambda i: (i, 0)),
                   pl.BlockSpec((BN, 16), lambda i: (i, 0))],
        out_shape=[jax.ShapeDtypeStruct((_NPAD, 128), _f32),
                   jax.ShapeDtypeStruct((_NPAD, 16), _f32),
                   jax.ShapeDtypeStruct((_NPAD, 128), _f32),
                   jax.ShapeDtypeStruct((_NPAD, 16), _f32)])(h, W)


def _node_call(h, aggs, ea0, ea1, na, Wself, WedgeP, Wattr, Wmsg_next,
               *, cat, do_silu, split_next):
    BN = 128
    d1 = Wself.shape[1]
    nagg = len(aggs)

    def body(h_ref, *rest):
        agg_refs = rest[:nagg]
        e0_ref, e1_ref, na_ref, ws_ref, we_ref, wa_ref = rest[nagg:nagg + 6]
        rest = rest[nagg + 6:]
        if Wmsg_next is not None:
            wm_ref, outs = rest[0], rest[1:]
        else:
            outs = rest
        if cat:
            agg = _cat4(*(r[...] for r in agg_refs))
        else:
            agg = agg_refs[0][...] + agg_refs[1][...]
        ea = e0_ref[...] + e1_ref[...]
        z = jnp.dot(h_ref[...], ws_ref[...], preferred_element_type=_f32)
        z = z + (agg + jnp.dot(ea, we_ref[...], preferred_element_type=_f32)) * _INV
        z = z + na_ref[...][:, :1] * wa_ref[...]
        if do_silu:
            z = _silu(z)
        outs[0][...] = z
        if Wmsg_next is not None:
            hm = jnp.dot(z, wm_ref[...], preferred_element_type=_f32)
            if split_next:
                m0, t0, m1, t1 = _split4(hm)
                outs[1][...], outs[2][...], outs[3][...], outs[4][...] = \
                    m0, t0, m1, t1
            else:
                outs[1][...] = hm

    def full(shp):
        return pl.BlockSpec(shp, lambda i: (0, 0))

    def rowblk(w):
        return pl.BlockSpec((BN, w), lambda i: (i, 0))

    in_specs = [rowblk(h.shape[1])]
    in_specs += [rowblk(a.shape[1]) for a in aggs]
    in_specs += [rowblk(16), rowblk(16), rowblk(16),
                 full(Wself.shape), full(WedgeP.shape), full(Wattr.shape)]
    args = [h, *aggs, ea0, ea1, na, Wself, WedgeP, Wattr]
    out_shape = [jax.ShapeDtypeStruct((_NPAD, d1), _f32)]
    out_specs = [rowblk(d1)]
    if Wmsg_next is not None:
        in_specs.append(full(Wmsg_next.shape))
        args.append(Wmsg_next)
        if split_next:
            out_shape += [jax.ShapeDtypeStruct((_NPAD, 128), _f32),
                          jax.ShapeDtypeStruct((_NPAD, 16), _f32),
                          jax.ShapeDtypeStruct((_NPAD, 128), _f32),
                          jax.ShapeDtypeStruct((_NPAD, 16), _f32)]
            out_specs += [rowblk(128), rowblk(16), rowblk(128), rowblk(16)]
        else:
            dn = Wmsg_next.shape[1]
            out_shape.append(jax.ShapeDtypeStruct((_NPAD, dn), _f32))
            out_specs.append(rowblk(dn))
    return pl.pallas_call(
        body, grid=(_NPAD // BN,), in_specs=in_specs, out_specs=out_specs,
        out_shape=out_shape)(*args)


def _softmax_call(pool):
    def body(p_ref, o_ref):
        p = p_ref[0] + p_ref[1]
        s = p[:_NG, :10]
        m = jnp.max(s, axis=1, keepdims=True)
        e = jnp.exp(s - m)
        o_ref[...] = e / jnp.sum(e, axis=1, keepdims=True)

    return pl.pallas_call(
        body, out_shape=jax.ShapeDtypeStruct((_NG, 10), _f32))(pool)


def _zero_buf(buf, ncols):
    z16 = jnp.zeros((16,), _f32)

    def zrow(i, cc):
        for j in range(ncols // 16):
            buf[i, pl.ds(j * 16, 16)] = z16
        return cc
    lax.fori_loop(0, buf.shape[0], zrow, 0)


def _tiled_copy(src_getter, dst_getter, nrows):
    full_copies = _RPT // nrows
    rem = _RPT - full_copies * nrows
    for t in range(full_copies):
        pltpu.sync_copy(src_getter(t * nrows, nrows), dst_getter(t * nrows, nrows))
    if rem:
        off = full_copies * nrows
        pltpu.sync_copy(src_getter(off, rem), dst_getter(off, rem))


def _edge_fs_call(hms, ws, src2, dst2):
    def body(hm0_ref, ht0_ref, hm1_ref, ht1_ref,
             wm0_ref, wt0_ref, wm1_ref, wt1_ref, src_ref, dst_ref,
             om0_ref, ot0_ref, om1_ref, ot1_ref,
             sidx, didx, rowm, rowt, wbm, wbt, accm, acct,
             semg0, semg1, semw0, semw1, sems):
        c = lax.axis_index("c")
        s = lax.axis_index("s")
        tile_row = s * _NCH
        tile_edge = s * _NCH * _CF

        _zero_buf(rowm.at[0], 128)
        _zero_buf(rowt.at[0], 16)
        _tiled_copy(lambda o, n: rowm.at[0].at[pl.ds(0, n)],
                    lambda o, n: accm.at[pl.ds(s * _RPT + o, n)], _CF)
        _tiled_copy(lambda o, n: rowt.at[0].at[pl.ds(0, n)],
                    lambda o, n: acct.at[pl.ds(s * _RPT + o, n)], _CF)
        plsc.subcore_barrier()

        semg = (semg0, semg1)
        semw = (semw0, semw1)

        def main(hm_ref, ht_ref, wm_ref, wt_ref, om_ref, ot_ref):
            def g_start(buf, ib, islot):
                pltpu.async_copy(hm_ref.at[sidx.at[ib, islot]], rowm.at[buf],
                                 semg[buf])
                pltpu.async_copy(ht_ref.at[sidx.at[ib, islot]], rowt.at[buf],
                                 semg[buf])

            def g_wait(buf):
                pltpu.make_async_copy(
                    hm_ref.at[sidx.at[0, 0]], rowm.at[buf], semg[buf]).wait()
                pltpu.make_async_copy(
                    ht_ref.at[sidx.at[0, 0]], rowt.at[buf], semg[buf]).wait()

            def w_start(ebase, buf):
                pltpu.async_copy(wm_ref.at[pl.ds(ebase, _CF)], wbm.at[buf],
                                 semw[buf])
                pltpu.async_copy(wt_ref.at[pl.ds(ebase, _CF)], wbt.at[buf],
                                 semw[buf])

            def w_wait(buf):
                pltpu.make_async_copy(
                    wm_ref.at[pl.ds(tile_edge, _CF)], wbm.at[buf], semw[buf]).wait()
                pltpu.make_async_copy(
                    wt_ref.at[pl.ds(tile_edge, _CF)], wbt.at[buf], semw[buf]).wait()

            def s_start(buf, ib, islot):
                pltpu.async_copy(rowm.at[buf], accm.at[didx.at[ib, islot]],
                                 sems, add=True)
                pltpu.async_copy(rowt.at[buf], acct.at[didx.at[ib, islot]],
                                 sems, add=True)

            def s_wait():
                pltpu.make_async_copy(
                    rowm.at[0], accm.at[didx.at[0, 0]], sems).wait()
                pltpu.make_async_copy(
                    rowt.at[0], acct.at[didx.at[0, 0]], sems).wait()

            def mul(buf):
                def mrow(i, c2):
                    for j in range(8):
                        sl = pl.ds(j * 16, 16)
                        rowm[buf, i, sl] = rowm[buf, i, sl] * wbm[buf, i, sl]
                    sl = pl.ds(0, 16)
                    rowt[buf, i, sl] = rowt[buf, i, sl] * wbt[buf, i, sl]
                    return c2
                lax.fori_loop(0, _CF, mrow, 0)

            pltpu.sync_copy(src_ref.at[pl.ds(tile_row, 8)], sidx.at[0])
            pltpu.sync_copy(dst_ref.at[pl.ds(tile_row, 8)], didx.at[0])
            g_start(0, 0, 0)
            w_start(tile_edge, 0)

            def pair(p, cc):
                a = 2 * p
                b = a + 1
                q = lax.rem(p, 4)
                grp = lax.div(p, 4)

                @pl.when(jnp.logical_and(q == 0, grp + 1 < _NGRP))
                def _():
                    row = tile_row + (grp + 1) * 8
                    pltpu.sync_copy(src_ref.at[pl.ds(row, 8)],
                                    sidx.at[lax.rem(grp + 1, 2)])
                    pltpu.sync_copy(dst_ref.at[pl.ds(row, 8)],
                                    didx.at[lax.rem(grp + 1, 2)])

                ib = lax.rem(grp, 2)

                @pl.when(p > 0)
                def _():
                    s_wait()
                g_start(1, ib, lax.rem(b, 8))
                w_start(tile_edge + b * _CF, 1)
                g_wait(0)
                w_wait(0)
                mul(0)
                s_start(0, ib, lax.rem(a, 8))

                s_wait()
                @pl.when(p + 1 < _NPAIR)
                def _():
                    nk = a + 2
                    g_start(0, lax.rem(lax.div(nk, 8), 2), lax.rem(nk, 8))
                    w_start(tile_edge + nk * _CF, 0)
                g_wait(1)
                w_wait(1)
                mul(1)
                s_start(1, ib, lax.rem(b, 8))
                return cc

            lax.fori_loop(0, _NPAIR, pair, 0)
            s_wait()
            plsc.subcore_barrier()
            _tiled_copy(lambda o, n: accm.at[pl.ds(s * _RPT + o, n)],
                        lambda o, n: om_ref.at[pl.ds(s * _RPT + o, n)], 128)
            _tiled_copy(lambda o, n: acct.at[pl.ds(s * _RPT + o, n)],
                        lambda o, n: ot_ref.at[pl.ds(s * _RPT + o, n)], 128)

        @pl.when(c == 0)
        def _():
            main(hm0_ref, ht0_ref, wm0_ref, wt0_ref, om0_ref, ot0_ref)

        @pl.when(c == 1)
        def _():
            main(hm1_ref, ht1_ref, wm1_ref, wt1_ref, om1_ref, ot1_ref)

    return pl.kernel(
        body,
        out_type=[jax.ShapeDtypeStruct((_NPAD, 128), _f32),
                  jax.ShapeDtypeStruct((_NPAD, 16), _f32),
                  jax.ShapeDtypeStruct((_NPAD, 128), _f32),
                  jax.ShapeDtypeStruct((_NPAD, 16), _f32)],
        mesh=_MESH,
        compiler_params=_SC_PARAMS,
        scratch_types=[
            pltpu.VMEM((2, 8, _CF), jnp.int32),
            pltpu.VMEM((2, 8, _CF), jnp.int32),
            pltpu.VMEM((2, _CF, 128), _f32),
            pltpu.VMEM((2, _CF, 16), _f32),
            pltpu.VMEM((2, _CF, 128), _f32),
            pltpu.VMEM((2, _CF, 16), _f32),
            pltpu.VMEM_SHARED((_NPAD, 128), _f32),
            pltpu.VMEM_SHARED((_NPAD, 16), _f32),
            pltpu.SemaphoreType.DMA,
            pltpu.SemaphoreType.DMA,
            pltpu.SemaphoreType.DMA,
            pltpu.SemaphoreType.DMA,
            pltpu.SemaphoreType.DMA,
        ])(*hms, *ws, src2, dst2)


def _edge_es_call(hm4, w4, srcp, dstp):
    NCH = _EPAD // (32 * _C)

    def body(hm_ref, w_ref, src_ref, dst_ref, out_ref,
             sidx, didx, rows, wbuf, acc, sem):
        c = lax.axis_index("c")
        s = lax.axis_index("s")
        wid = c * 16 + s

        _zero_buf(rows, 16)
        _tiled_copy(lambda o, n: rows.at[pl.ds(0, n)],
                    lambda o, n: acc.at[pl.ds(s * _RPT + o, n)], _C)
        plsc.subcore_barrier()

        def chunk(k, cc):
            base = (wid * NCH + k) * _C
            pltpu.sync_copy(src_ref.at[pl.ds(base, _C)], sidx)
            pltpu.sync_copy(dst_ref.at[pl.ds(base, _C)], didx)
            cp = pltpu.async_copy(hm_ref.at[sidx], rows, sem)
            pltpu.sync_copy(w_ref.at[pl.ds(base, _C)], wbuf)
            cp.wait()

            def mrow(i, c2):
                sl = pl.ds(0, 16)
                rows[i, sl] = rows[i, sl] * wbuf[i, sl]
                return c2
            lax.fori_loop(0, _C, mrow, 0)
            pltpu.sync_copy(rows, acc.at[didx], add=True)
            return cc
        lax.fori_loop(0, NCH, chunk, 0)
        plsc.subcore_barrier()
        _tiled_copy(lambda o, n: acc.at[pl.ds(s * _RPT + o, n)],
                    lambda o, n: out_ref.at[c, pl.ds(s * _RPT + o, n)], 128)

    return pl.kernel(
        body,
        out_type=jax.ShapeDtypeStruct((2, _NPAD, 16), _f32),
        mesh=_MESH,
        compiler_params=_SC_PARAMS,
        scratch_types=[
            pltpu.VMEM((_C,), jnp.int32),
            pltpu.VMEM((_C,), jnp.int32),
            pltpu.VMEM((_C, 16), _f32),
            pltpu.VMEM((_C, 16), _f32),
            pltpu.VMEM_SHARED((_NPAD, 16), _f32),
            pltpu.SemaphoreType.DMA,
        ])(hm4, w4, srcp, dstp)


def _ea_call(eap, dstp):
    NCH = _EPAD // (32 * _C)

    def body(ea_ref, dst_ref, out_ref, didx, rows, acc, sem):
        c = lax.axis_index("c")
        s = lax.axis_index("s")
        wid = c * 16 + s

        _zero_buf(rows, 16)
        _tiled_copy(lambda o, n: rows.at[pl.ds(0, n)],
                    lambda o, n: acc.at[pl.ds(s * _RPT + o, n)], _C)
        plsc.subcore_barrier()

        def chunk(k, cc):
            base = (wid * NCH + k) * _C
            pltpu.sync_copy(dst_ref.at[pl.ds(base, _C)], didx)
            pltpu.sync_copy(ea_ref.at[pl.ds(base, _C)], rows)
            pltpu.sync_copy(rows, acc.at[didx], add=True)
            return cc
        lax.fori_loop(0, NCH, chunk, 0)
        plsc.subcore_barrier()
        _tiled_copy(lambda o, n: acc.at[pl.ds(s * _RPT + o, n)],
                    lambda o, n: out_ref.at[c, pl.ds(s * _RPT + o, n)], 128)

    return pl.kernel(
        body,
        out_type=jax.ShapeDtypeStruct((2, _NPAD, 16), _f32),
        mesh=_MESH,
        compiler_params=_SC_PARAMS,
        scratch_types=[
            pltpu.VMEM((_C,), jnp.int32),
            pltpu.VMEM((_C, 16), _f32),
            pltpu.VMEM_SHARED((_NPAD, 16), _f32),
            pltpu.SemaphoreType.DMA,
        ])(eap, dstp)


def _pool_call(h4, batchp):
    CP = 64
    NCHT = _NPAD // CP

    def body(h_ref, b_ref, out_ref, bidx, rows, zbuf, acc, sem):
        c = lax.axis_index("c")
        s = lax.axis_index("s")
        wid = c * 16 + s

        @pl.when(s == 0)
        def _():
            _zero_buf(zbuf, 16)
            pltpu.sync_copy(zbuf, acc)
        plsc.subcore_barrier()

        def chunk(k, cc):
            idx = k * 32 + wid

            @pl.when(idx < NCHT)
            def _():
                base = idx * CP
                pltpu.sync_copy(b_ref.at[pl.ds(base, CP)], bidx)
                pltpu.sync_copy(h_ref.at[pl.ds(base, CP)], rows)
                pltpu.sync_copy(rows, acc.at[bidx], add=True)
            return cc
        lax.fori_loop(0, (NCHT + 31) // 32, chunk, 0)
        plsc.subcore_barrier()

        @pl.when(s == 0)
        def _():
            pltpu.sync_copy(acc, out_ref.at[c])

    return pl.kernel(
        body,
        out_type=jax.ShapeDtypeStruct((2, 72, 16), _f32),
        mesh=_MESH,
        compiler_params=_SC_PARAMS,
        scratch_types=[
            pltpu.VMEM((CP,), jnp.int32),
            pltpu.VMEM((CP, 16), _f32),
            pltpu.VMEM((72, 16), _f32),
            pltpu.VMEM_SHARED((72, 16), _f32),
            pltpu.SemaphoreType.DMA,
        ])(h4, batchp)


def kernel(x, node_attr, edge_src, edge_dst, edge_attr, edge_length_embedding,
           batch, params):
    xp = jnp.zeros((_NPAD, 128), _f32).at[:_N].set(x)
    nap = jnp.zeros((_NPAD, 16), _f32).at[:_N].set(
        jnp.broadcast_to(node_attr, (_N, 16)))
    srcp = jnp.full((_EPAD,), _N, jnp.int32).at[:_E].set(edge_src.astype(jnp.int32))
    dstp = jnp.full((_EPAD,), _N, jnp.int32).at[:_E].set(edge_dst.astype(jnp.int32))
    src2 = srcp.reshape(_EPAD // _CF, _CF)
    dst2 = dstp.reshape(_EPAD // _CF, _CF)
    eap = jnp.zeros((_EPAD, 16), _f32).at[:_E, :9].set(edge_attr)
    embT = jnp.zeros((10, _EPAD), _f32).at[:, :_E].set(edge_length_embedding.T)
    batchp = jnp.full((_NPAD,), _NG, jnp.int32).at[:_N].set(batch.astype(jnp.int32))

    def wedgeP(p):
        return jnp.zeros((16, p['W_edge'].shape[1]), _f32).at[:9].set(p['W_edge'])

    p4 = params[3]
    Wself4 = jnp.zeros((288, 16), _f32).at[:, :10].set(p4['W_self'])
    Wedge4 = jnp.zeros((16, 16), _f32).at[:9, :10].set(p4['W_edge'])
    Wattr4 = jnp.zeros((1, 16), _f32).at[:, :10].set(p4['W_attr'])
    fcW3_4 = jnp.zeros((128, 16), _f32).at[:, :10].set(p4['fc_W3'])
    Wmsg4 = jnp.zeros((288, 16), _f32).at[:, :10].set(p4['W_msg'])

    all_ws = [_radial_call(embT, params[li]['fc_W0'], params[li]['fc_W1'],
                           params[li]['fc_W2'], params[li]['fc_W3'], split=True)
              for li in range(3)]
    (w4,) = _radial_call(embT, p4['fc_W0'], p4['fc_W1'], p4['fc_W2'], fcW3_4,
                         split=False)
    ea_pair = _ea_call(eap, dstp)
    ea0, ea1 = ea_pair[0], ea_pair[1]

    hms = _msg_split_call(xp, params[0]['W_msg'])
    h = xp
    for li in range(3):
        p = params[li]
        ws = all_ws[li]
        aggs = _edge_fs_call(hms, ws, src2, dst2)
        if li < 2:
            h, *hms = _node_call(
                h, aggs, ea0, ea1, nap, p['W_self'], wedgeP(p),
                p['W_attr'], params[li + 1]['W_msg'],
                cat=True, do_silu=True, split_next=True)
        else:
            h, hm4 = _node_call(
                h, aggs, ea0, ea1, nap, p['W_self'], wedgeP(p),
                p['W_attr'], Wmsg4, cat=True, do_silu=True,
                split_next=False)

    agg4 = _edge_es_call(hm4, w4, srcp, dstp)
    (h4,) = _node_call(h, [agg4[0], agg4[1]], ea0, ea1, nap, Wself4, Wedge4,
                       Wattr4, None, cat=False, do_silu=False, split_next=False)
    pool = _pool_call(h4, batchp)
    return _softmax_call(pool)


# speedup vs baseline: 1.1062x; 1.0070x over previous
import numpy as np
import jax
import jax.numpy as jnp
from jax import lax
from jax.experimental import pallas as pl
from jax.experimental.pallas import tpu as pltpu
from jax.experimental.pallas import tpu_sc as plsc

_N = 10000
_NG = 64
_NPAD = 10112
_E = 320000
_EPAD = 327680
_C = 128
_CF = 64
_RPT = _NPAD // 16
_NCH = _EPAD // (16 * _CF)
_NPAIR = _NCH // 2
_NGRP = _NCH // 8
_INV = 1.0 / np.sqrt(32.0)

_MESH = plsc.VectorSubcoreMesh(
    core_axis_name="c", subcore_axis_name="s", num_cores=2, num_subcores=16)
_SC_PARAMS = pltpu.CompilerParams(use_tc_tiling_on_sc=False,
                                  needs_layout_passes=False)

_f32 = jnp.float32


def _silu(v):
    return v * lax.logistic(v)


def _split4(v):
    return v[:, :128], v[:, 128:144], v[:, 160:288], v[:, 144:160]


def _cat4(m0, t0, m1, t1):
    return jnp.concatenate([m0, t0, t1, m1], axis=1)


def _radial_call(embT, W0, W1, W2, W3, split):
    BE = 1024
    d1 = W3.shape[1]

    def body(emb_ref, w0_ref, w1_ref, w2_ref, w3_ref, *outs):
        bf = jnp.bfloat16
        v = _silu(lax.dot_general(
            emb_ref[...], w0_ref[...], (((0,), (0,)), ((), ())),
            preferred_element_type=_f32))
        v = _silu(jnp.dot(v.astype(bf), w1_ref[...].astype(bf),
                          preferred_element_type=_f32))
        v = _silu(jnp.dot(v.astype(bf), w2_ref[...].astype(bf),
                          preferred_element_type=_f32))
        v = jnp.dot(v.astype(bf), w3_ref[...].astype(bf),
                    preferred_element_type=_f32)
        if split:
            m0, t0, m1, t1 = _split4(v)
            outs[0][...], outs[1][...], outs[2][...], outs[3][...] = m0, t0, m1, t1
        else:
            outs[0][...] = v

    def full(shp):
        return pl.BlockSpec(shp, lambda i: (0, 0))

    in_specs = [pl.BlockSpec((embT.shape[0], BE), lambda i: (0, i)),
                full(W0.shape), full(W1.shape), full(W2.shape), full(W3.shape)]
    if split:
        out_shape = [jax.ShapeDtypeStruct((_EPAD, 128), _f32),
                     jax.ShapeDtypeStruct((_EPAD, 16), _f32),
                     jax.ShapeDtypeStruct((_EPAD, 128), _f32),
                     jax.ShapeDtypeStruct((_EPAD, 16), _f32)]
        out_specs = [pl.BlockSpec((BE, 128), lambda i: (i, 0)),
                     pl.BlockSpec((BE, 16), lambda i: (i, 0)),
                     pl.BlockSpec((BE, 128), lambda i: (i, 0)),
                     pl.BlockSpec((BE, 16), lambda i: (i, 0))]
    else:
        out_shape = [jax.ShapeDtypeStruct((_EPAD, d1), _f32)]
        out_specs = [pl.BlockSpec((BE, d1), lambda i: (i, 0))]
    return pl.pallas_call(
        body, grid=(_EPAD // BE,), in_specs=in_specs, out_specs=out_specs,
        out_shape=out_shape)(embT, W0, W1, W2, W3)


def _msg_split_call(h, W):
    BN = 128

    def body(h_ref, w_ref, o0, o1, o2, o3):
        hm = jnp.dot(h_ref[...], w_ref[...], preferred_element_type=_f32)
        m0, t0, m1, t1 = _split4(hm)
        o0[...], o1[...], o2[...], o3[...] = m0, t0, m1, t1

    return pl.pallas_call(
        body, grid=(_NPAD // BN,),
        in_specs=[pl.BlockSpec((BN, h.shape[1]), lambda i: (i, 0)),
                  pl.BlockSpec(W.shape, lambda i: (0, 0))],
        out_specs=[pl.BlockSpec((BN, 128), lambda i: (i, 0)),
                   pl.BlockSpec((BN, 16), lambda i: (i, 0)),
                   pl.BlockSpec((BN, 128), lambda i: (i, 0)),
                   pl.BlockSpec((BN, 16), lambda i: (i, 0))],
        out_shape=[jax.ShapeDtypeStruct((_NPAD, 128), _f32),
                   jax.ShapeDtypeStruct((_NPAD, 16), _f32),
                   jax.ShapeDtypeStruct((_NPAD, 128), _f32),
                   jax.ShapeDtypeStruct((_NPAD, 16), _f32)])(h, W)


def _node_call(h, aggs, ea0, ea1, na, Wself, WedgeP, Wattr, Wmsg_next,
               *, cat, do_silu, split_next):
    BN = 128
    d1 = Wself.shape[1]
    nagg = len(aggs)

    def body(h_ref, *rest):
        agg_refs = rest[:nagg]
        e0_ref, e1_ref, na_ref, ws_ref, we_ref, wa_ref = rest[nagg:nagg + 6]
        rest = rest[nagg + 6:]
        if Wmsg_next is not None:
            wm_ref, outs = rest[0], rest[1:]
        else:
            outs = rest
        if cat:
            agg = _cat4(*(r[...] for r in agg_refs))
        else:
            agg = agg_refs[0][...] + agg_refs[1][...]
        ea = e0_ref[...] + e1_ref[...]
        z = jnp.dot(h_ref[...], ws_ref[...], preferred_element_type=_f32)
        z = z + (agg + jnp.dot(ea, we_ref[...], preferred_element_type=_f32)) * _INV
        z = z + na_ref[...][:, :1] * wa_ref[...]
        if do_silu:
            z = _silu(z)
        outs[0][...] = z
        if Wmsg_next is not None:
            hm = jnp.dot(z, wm_ref[...], preferred_element_type=_f32)
            if split_next:
                m0, t0, m1, t1 = _split4(hm)
                outs[1][...], outs[2][...], outs[3][...], outs[4][...] = \
                    m0, t0, m1, t1
            else:
                outs[1][...] = hm

    def full(shp):
        return pl.BlockSpec(shp, lambda i: (0, 0))

    def rowblk(w):
        return pl.BlockSpec((BN, w), lambda i: (i, 0))

    in_specs = [rowblk(h.shape[1])]
    in_specs += [rowblk(a.shape[1]) for a in aggs]
    in_specs += [rowblk(16), rowblk(16), rowblk(16),
                 full(Wself.shape), full(WedgeP.shape), full(Wattr.shape)]
    args = [h, *aggs, ea0, ea1, na, Wself, WedgeP, Wattr]
    out_shape = [jax.ShapeDtypeStruct((_NPAD, d1), _f32)]
    out_specs = [rowblk(d1)]
    if Wmsg_next is not None:
        in_specs.append(full(Wmsg_next.shape))
        args.append(Wmsg_next)
        if split_next:
            out_shape += [jax.ShapeDtypeStruct((_NPAD, 128), _f32),
                          jax.ShapeDtypeStruct((_NPAD, 16), _f32),
                          jax.ShapeDtypeStruct((_NPAD, 128), _f32),
                          jax.ShapeDtypeStruct((_NPAD, 16), _f32)]
            out_specs += [rowblk(128), rowblk(16), rowblk(128), rowblk(16)]
        else:
            dn = Wmsg_next.shape[1]
            out_shape.append(jax.ShapeDtypeStruct((_NPAD, dn), _f32))
            out_specs.append(rowblk(dn))
    return pl.pallas_call(
        body, grid=(_NPAD // BN,), in_specs=in_specs, out_specs=out_specs,
        out_shape=out_shape)(*args)


def _softmax_call(pool):
    def body(p_ref, o_ref):
        p = p_ref[0] + p_ref[1]
        s = p[:_NG, :10]
        m = jnp.max(s, axis=1, keepdims=True)
        e = jnp.exp(s - m)
        o_ref[...] = e / jnp.sum(e, axis=1, keepdims=True)

    return pl.pallas_call(
        body, out_shape=jax.ShapeDtypeStruct((_NG, 10), _f32))(pool)


def _zero_buf(buf, ncols):
    z16 = jnp.zeros((16,), _f32)

    def zrow(i, cc):
        for j in range(ncols // 16):
            buf[i, pl.ds(j * 16, 16)] = z16
        return cc
    lax.fori_loop(0, buf.shape[0], zrow, 0)


def _tiled_copy(src_getter, dst_getter, nrows):
    full_copies = _RPT // nrows
    rem = _RPT - full_copies * nrows
    for t in range(full_copies):
        pltpu.sync_copy(src_getter(t * nrows, nrows), dst_getter(t * nrows, nrows))
    if rem:
        off = full_copies * nrows
        pltpu.sync_copy(src_getter(off, rem), dst_getter(off, rem))


def _edge_fs_call(hms, ws, src2, dst2):
    def body(hm0_ref, ht0_ref, hm1_ref, ht1_ref,
             wm0_ref, wt0_ref, wm1_ref, wt1_ref, src_ref, dst_ref,
             om0_ref, ot0_ref, om1_ref, ot1_ref,
             sidx, didx, rowm, rowt, wbm, wbt, accm, acct,
             semg0, semg1, semw0, semw1, sems):
        c = lax.axis_index("c")
        s = lax.axis_index("s")
        tile_row = s * _NCH
        tile_edge = s * _NCH * _CF

        _zero_buf(rowm.at[0], 128)
        _zero_buf(rowt.at[0], 16)
        _tiled_copy(lambda o, n: rowm.at[0].at[pl.ds(0, n)],
                    lambda o, n: accm.at[pl.ds(s * _RPT + o, n)], _CF)
        _tiled_copy(lambda o, n: rowt.at[0].at[pl.ds(0, n)],
                    lambda o, n: acct.at[pl.ds(s * _RPT + o, n)], _CF)
        plsc.subcore_barrier()

        semg = (semg0, semg1)
        semw = (semw0, semw1)

        def main(hm_ref, ht_ref, wm_ref, wt_ref, om_ref, ot_ref):
            def g_start(buf, ib, islot):
                pltpu.async_copy(hm_ref.at[sidx.at[ib, islot]], rowm.at[buf],
                                 semg[buf])
                pltpu.async_copy(ht_ref.at[sidx.at[ib, islot]], rowt.at[buf],
                                 semg[buf])

            def g_wait(buf):
                pltpu.make_async_copy(
                    hm_ref.at[sidx.at[0, 0]], rowm.at[buf], semg[buf]).wait()
                pltpu.make_async_copy(
                    ht_ref.at[sidx.at[0, 0]], rowt.at[buf], semg[buf]).wait()

            def w_start(ebase, buf):
                pltpu.async_copy(wm_ref.at[pl.ds(ebase, _CF)], wbm.at[buf],
                                 semw[buf])
                pltpu.async_copy(wt_ref.at[pl.ds(ebase, _CF)], wbt.at[buf],
                                 semw[buf])

            def w_wait(buf):
                pltpu.make_async_copy(
                    wm_ref.at[pl.ds(tile_edge, _CF)], wbm.at[buf], semw[buf]).wait()
                pltpu.make_async_copy(
                    wt_ref.at[pl.ds(tile_edge, _CF)], wbt.at[buf], semw[buf]).wait()

            def s_start(buf, ib, islot):
                pltpu.async_copy(rowm.at[buf], accm.at[didx.at[ib, islot]],
                                 sems, add=True)
                pltpu.async_copy(rowt.at[buf], acct.at[didx.at[ib, islot]],
                                 sems, add=True)

            def s_wait():
                pltpu.make_async_copy(
                    rowm.at[0], accm.at[didx.at[0, 0]], sems).wait()
                pltpu.make_async_copy(
                    rowt.at[0], acct.at[didx.at[0, 0]], sems).wait()

            def mul(buf):
                def mrow(i2, c2):
                    for u in range(2):
                        i = 2 * i2 + u
                        for j in range(8):
                            sl = pl.ds(j * 16, 16)
                            rowm[buf, i, sl] = rowm[buf, i, sl] * wbm[buf, i, sl]
                        sl = pl.ds(0, 16)
                        rowt[buf, i, sl] = rowt[buf, i, sl] * wbt[buf, i, sl]
                    return c2
                lax.fori_loop(0, _CF // 2, mrow, 0)

            pltpu.sync_copy(src_ref.at[pl.ds(tile_row, 8)], sidx.at[0])
            pltpu.sync_copy(dst_ref.at[pl.ds(tile_row, 8)], didx.at[0])
            g_start(0, 0, 0)
            w_start(tile_edge, 0)

            def pair(p, cc):
                a = 2 * p
                b = a + 1
                q = lax.rem(p, 4)
                grp = lax.div(p, 4)

                @pl.when(jnp.logical_and(q == 0, grp + 1 < _NGRP))
                def _():
                    row = tile_row + (grp + 1) * 8
                    pltpu.sync_copy(src_ref.at[pl.ds(row, 8)],
                                    sidx.at[lax.rem(grp + 1, 2)])
                    pltpu.sync_copy(dst_ref.at[pl.ds(row, 8)],
                                    didx.at[lax.rem(grp + 1, 2)])

                ib = lax.rem(grp, 2)

                @pl.when(p > 0)
                def _():
                    s_wait()
                g_start(1, ib, lax.rem(b, 8))
                w_start(tile_edge + b * _CF, 1)
                g_wait(0)
                w_wait(0)
                mul(0)
                s_start(0, ib, lax.rem(a, 8))

                s_wait()
                @pl.when(p + 1 < _NPAIR)
                def _():
                    nk = a + 2
                    g_start(0, lax.rem(lax.div(nk, 8), 2), lax.rem(nk, 8))
                    w_start(tile_edge + nk * _CF, 0)
                g_wait(1)
                w_wait(1)
                mul(1)
                s_start(1, ib, lax.rem(b, 8))
                return cc

            lax.fori_loop(0, _NPAIR, pair, 0)
            s_wait()
            plsc.subcore_barrier()
            _tiled_copy(lambda o, n: accm.at[pl.ds(s * _RPT + o, n)],
                        lambda o, n: om_ref.at[pl.ds(s * _RPT + o, n)], 128)
            _tiled_copy(lambda o, n: acct.at[pl.ds(s * _RPT + o, n)],
                        lambda o, n: ot_ref.at[pl.ds(s * _RPT + o, n)], 128)

        @pl.when(c == 0)
        def _():
            main(hm0_ref, ht0_ref, wm0_ref, wt0_ref, om0_ref, ot0_ref)

        @pl.when(c == 1)
        def _():
            main(hm1_ref, ht1_ref, wm1_ref, wt1_ref, om1_ref, ot1_ref)

    return pl.kernel(
        body,
        out_type=[jax.ShapeDtypeStruct((_NPAD, 128), _f32),
                  jax.ShapeDtypeStruct((_NPAD, 16), _f32),
                  jax.ShapeDtypeStruct((_NPAD, 128), _f32),
                  jax.ShapeDtypeStruct((_NPAD, 16), _f32)],
        mesh=_MESH,
        compiler_params=_SC_PARAMS,
        scratch_types=[
            pltpu.VMEM((2, 8, _CF), jnp.int32),
            pltpu.VMEM((2, 8, _CF), jnp.int32),
            pltpu.VMEM((2, _CF, 128), _f32),
            pltpu.VMEM((2, _CF, 16), _f32),
            pltpu.VMEM((2, _CF, 128), _f32),
            pltpu.VMEM((2, _CF, 16), _f32),
            pltpu.VMEM_SHARED((_NPAD, 128), _f32),
            pltpu.VMEM_SHARED((_NPAD, 16), _f32),
            pltpu.SemaphoreType.DMA,
            pltpu.SemaphoreType.DMA,
            pltpu.SemaphoreType.DMA,
            pltpu.SemaphoreType.DMA,
            pltpu.SemaphoreType.DMA,
        ])(*hms, *ws, src2, dst2)


def _edge_es_call(hm4, w4, srcp, dstp):
    NCH = _EPAD // (32 * _C)

    def body(hm_ref, w_ref, src_ref, dst_ref, out_ref,
             sidx, didx, rows, wbuf, acc, sem):
        c = lax.axis_index("c")
        s = lax.axis_index("s")
        wid = c * 16 + s

        _zero_buf(rows, 16)
        _tiled_copy(lambda o, n: rows.at[pl.ds(0, n)],
                    lambda o, n: acc.at[pl.ds(s * _RPT + o, n)], _C)
        plsc.subcore_barrier()

        def chunk(k, cc):
            base = (wid * NCH + k) * _C
            pltpu.sync_copy(src_ref.at[pl.ds(base, _C)], sidx)
            pltpu.sync_copy(dst_ref.at[pl.ds(base, _C)], didx)
            cp = pltpu.async_copy(hm_ref.at[sidx], rows, sem)
            pltpu.sync_copy(w_ref.at[pl.ds(base, _C)], wbuf)
            cp.wait()

            def mrow(i, c2):
                sl = pl.ds(0, 16)
                rows[i, sl] = rows[i, sl] * wbuf[i, sl]
                return c2
            lax.fori_loop(0, _C, mrow, 0)
            pltpu.sync_copy(rows, acc.at[didx], add=True)
            return cc
        lax.fori_loop(0, NCH, chunk, 0)
        plsc.subcore_barrier()
        _tiled_copy(lambda o, n: acc.at[pl.ds(s * _RPT + o, n)],
                    lambda o, n: out_ref.at[c, pl.ds(s * _RPT + o, n)], 128)

    return pl.kernel(
        body,
        out_type=jax.ShapeDtypeStruct((2, _NPAD, 16), _f32),
        mesh=_MESH,
        compiler_params=_SC_PARAMS,
        scratch_types=[
            pltpu.VMEM((_C,), jnp.int32),
            pltpu.VMEM((_C,), jnp.int32),
            pltpu.VMEM((_C, 16), _f32),
            pltpu.VMEM((_C, 16), _f32),
            pltpu.VMEM_SHARED((_NPAD, 16), _f32),
            pltpu.SemaphoreType.DMA,
        ])(hm4, w4, srcp, dstp)


def _ea_call(eap, dstp):
    NCH = _EPAD // (32 * _C)

    def body(ea_ref, dst_ref, out_ref, didx, rows, acc, sem):
        c = lax.axis_index("c")
        s = lax.axis_index("s")
        wid = c * 16 + s

        _zero_buf(rows, 16)
        _tiled_copy(lambda o, n: rows.at[pl.ds(0, n)],
                    lambda o, n: acc.at[pl.ds(s * _RPT + o, n)], _C)
        plsc.subcore_barrier()

        def chunk(k, cc):
            base = (wid * NCH + k) * _C
            pltpu.sync_copy(dst_ref.at[pl.ds(base, _C)], didx)
            pltpu.sync_copy(ea_ref.at[pl.ds(base, _C)], rows)
            pltpu.sync_copy(rows, acc.at[didx], add=True)
            return cc
        lax.fori_loop(0, NCH, chunk, 0)
        plsc.subcore_barrier()
        _tiled_copy(lambda o, n: acc.at[pl.ds(s * _RPT + o, n)],
                    lambda o, n: out_ref.at[c, pl.ds(s * _RPT + o, n)], 128)

    return pl.kernel(
        body,
        out_type=jax.ShapeDtypeStruct((2, _NPAD, 16), _f32),
        mesh=_MESH,
        compiler_params=_SC_PARAMS,
        scratch_types=[
            pltpu.VMEM((_C,), jnp.int32),
            pltpu.VMEM((_C, 16), _f32),
            pltpu.VMEM_SHARED((_NPAD, 16), _f32),
            pltpu.SemaphoreType.DMA,
        ])(eap, dstp)


def _pool_call(h4, batchp):
    CP = 64
    NCHT = _NPAD // CP

    def body(h_ref, b_ref, out_ref, bidx, rows, zbuf, acc, sem):
        c = lax.axis_index("c")
        s = lax.axis_index("s")
        wid = c * 16 + s

        @pl.when(s == 0)
        def _():
            _zero_buf(zbuf, 16)
            pltpu.sync_copy(zbuf, acc)
        plsc.subcore_barrier()

        def chunk(k, cc):
            idx = k * 32 + wid

            @pl.when(idx < NCHT)
            def _():
                base = idx * CP
                pltpu.sync_copy(b_ref.at[pl.ds(base, CP)], bidx)
                pltpu.sync_copy(h_ref.at[pl.ds(base, CP)], rows)
                pltpu.sync_copy(rows, acc.at[bidx], add=True)
            return cc
        lax.fori_loop(0, (NCHT + 31) // 32, chunk, 0)
        plsc.subcore_barrier()

        @pl.when(s == 0)
        def _():
            pltpu.sync_copy(acc, out_ref.at[c])

    return pl.kernel(
        body,
        out_type=jax.ShapeDtypeStruct((2, 72, 16), _f32),
        mesh=_MESH,
        compiler_params=_SC_PARAMS,
        scratch_types=[
            pltpu.VMEM((CP,), jnp.int32),
            pltpu.VMEM((CP, 16), _f32),
            pltpu.VMEM((72, 16), _f32),
            pltpu.VMEM_SHARED((72, 16), _f32),
            pltpu.SemaphoreType.DMA,
        ])(h4, batchp)


def kernel(x, node_attr, edge_src, edge_dst, edge_attr, edge_length_embedding,
           batch, params):
    xp = jnp.zeros((_NPAD, 128), _f32).at[:_N].set(x)
    nap = jnp.zeros((_NPAD, 16), _f32).at[:_N].set(
        jnp.broadcast_to(node_attr, (_N, 16)))
    srcp = jnp.full((_EPAD,), _N, jnp.int32).at[:_E].set(edge_src.astype(jnp.int32))
    dstp = jnp.full((_EPAD,), _N, jnp.int32).at[:_E].set(edge_dst.astype(jnp.int32))
    src2 = srcp.reshape(_EPAD // _CF, _CF)
    dst2 = dstp.reshape(_EPAD // _CF, _CF)
    eap = jnp.zeros((_EPAD, 16), _f32).at[:_E, :9].set(edge_attr)
    embT = jnp.zeros((10, _EPAD), _f32).at[:, :_E].set(edge_length_embedding.T)
    batchp = jnp.full((_NPAD,), _NG, jnp.int32).at[:_N].set(batch.astype(jnp.int32))

    def wedgeP(p):
        return jnp.zeros((16, p['W_edge'].shape[1]), _f32).at[:9].set(p['W_edge'])

    p4 = params[3]
    Wself4 = jnp.zeros((288, 16), _f32).at[:, :10].set(p4['W_self'])
    Wedge4 = jnp.zeros((16, 16), _f32).at[:9, :10].set(p4['W_edge'])
    Wattr4 = jnp.zeros((1, 16), _f32).at[:, :10].set(p4['W_attr'])
    fcW3_4 = jnp.zeros((128, 16), _f32).at[:, :10].set(p4['fc_W3'])
    Wmsg4 = jnp.zeros((288, 16), _f32).at[:, :10].set(p4['W_msg'])

    ea_pair = _ea_call(eap, dstp)
    ea0, ea1 = ea_pair[0], ea_pair[1]
    all_ws = [_radial_call(embT, params[li]['fc_W0'], params[li]['fc_W1'],
                           params[li]['fc_W2'], params[li]['fc_W3'], split=True)
              for li in range(3)]
    (w4,) = _radial_call(embT, p4['fc_W0'], p4['fc_W1'], p4['fc_W2'], fcW3_4,
                         split=False)

    hms = _msg_split_call(xp, params[0]['W_msg'])
    h = xp
    for li in range(3):
        p = params[li]
        ws = all_ws[li]
        aggs = _edge_fs_call(hms, ws, src2, dst2)
        if li < 2:
            h, *hms = _node_call(
                h, aggs, ea0, ea1, nap, p['W_self'], wedgeP(p),
                p['W_attr'], params[li + 1]['W_msg'],
                cat=True, do_silu=True, split_next=True)
        else:
            h, hm4 = _node_call(
                h, aggs, ea0, ea1, nap, p['W_self'], wedgeP(p),
                p['W_attr'], Wmsg4, cat=True, do_silu=True,
                split_next=False)

    agg4 = _edge_es_call(hm4, w4, srcp, dstp)
    (h4,) = _node_call(h, [agg4[0], agg4[1]], ea0, ea1, nap, Wself4, Wedge4,
                       Wattr4, None, cat=False, do_silu=False, split_next=False)
    pool = _pool_call(h4, batchp)
    return _softmax_call(pool)


# speedup vs baseline: 1.1062x; 1.0000x over previous
import numpy as np
import jax
import jax.numpy as jnp
from jax import lax
from jax.experimental import pallas as pl
from jax.experimental.pallas import tpu as pltpu
from jax.experimental.pallas import tpu_sc as plsc

_N = 10000
_NG = 64
_NPAD = 10112
_E = 320000
_EPAD = 327680
_C = 128
_CF = 64
_RPT = _NPAD // 16
_NCH = _EPAD // (16 * _CF)
_NPAIR = _NCH // 2
_NGRP = _NCH // 8
_INV = 1.0 / np.sqrt(32.0)

_MESH = plsc.VectorSubcoreMesh(
    core_axis_name="c", subcore_axis_name="s", num_cores=2, num_subcores=16)
_SC_PARAMS = pltpu.CompilerParams(use_tc_tiling_on_sc=False,
                                  needs_layout_passes=False)

_f32 = jnp.float32


def _silu(v):
    return v * lax.logistic(v)


def _split4(v):
    return v[:, :128], v[:, 128:144], v[:, 160:288], v[:, 144:160]


def _cat4(m0, t0, m1, t1):
    return jnp.concatenate([m0, t0, t1, m1], axis=1)


def _radial_call(embT, W0, W1, W2, W3, split):
    BE = 1024
    d1 = W3.shape[1]

    def body(emb_ref, w0_ref, w1_ref, w2_ref, w3_ref, *outs):
        v = _silu(lax.dot_general(
            emb_ref[...], w0_ref[...], (((0,), (0,)), ((), ())),
            preferred_element_type=_f32))
        v = _silu(jnp.dot(v, w1_ref[...], preferred_element_type=_f32))
        v = _silu(jnp.dot(v, w2_ref[...], preferred_element_type=_f32))
        v = jnp.dot(v, w3_ref[...], preferred_element_type=_f32)
        if split:
            m0, t0, m1, t1 = _split4(v)
            outs[0][...], outs[1][...], outs[2][...], outs[3][...] = m0, t0, m1, t1
        else:
            outs[0][...] = v

    def full(shp):
        return pl.BlockSpec(shp, lambda i: (0, 0))

    in_specs = [pl.BlockSpec((embT.shape[0], BE), lambda i: (0, i)),
                full(W0.shape), full(W1.shape), full(W2.shape), full(W3.shape)]
    if split:
        out_shape = [jax.ShapeDtypeStruct((_EPAD, 128), _f32),
                     jax.ShapeDtypeStruct((_EPAD, 16), _f32),
                     jax.ShapeDtypeStruct((_EPAD, 128), _f32),
                     jax.ShapeDtypeStruct((_EPAD, 16), _f32)]
        out_specs = [pl.BlockSpec((BE, 128), lambda i: (i, 0)),
                     pl.BlockSpec((BE, 16), lambda i: (i, 0)),
                     pl.BlockSpec((BE, 128), lambda i: (i, 0)),
                     pl.BlockSpec((BE, 16), lambda i: (i, 0))]
    else:
        out_shape = [jax.ShapeDtypeStruct((_EPAD, d1), _f32)]
        out_specs = [pl.BlockSpec((BE, d1), lambda i: (i, 0))]
    return pl.pallas_call(
        body, grid=(_EPAD // BE,), in_specs=in_specs, out_specs=out_specs,
        out_shape=out_shape)(embT, W0, W1, W2, W3)


def _msg_split_call(h, W):
    BN = 128

    def body(h_ref, w_ref, o0, o1, o2, o3):
        hm = jnp.dot(h_ref[...], w_ref[...], preferred_element_type=_f32)
        m0, t0, m1, t1 = _split4(hm)
        o0[...], o1[...], o2[...], o3[...] = m0, t0, m1, t1

    return pl.pallas_call(
        body, grid=(_NPAD // BN,),
        in_specs=[pl.BlockSpec((BN, h.shape[1]), lambda i: (i, 0)),
                  pl.BlockSpec(W.shape, lambda i: (0, 0))],
        out_specs=[pl.BlockSpec((BN, 128), lambda i: (i, 0)),
                   pl.BlockSpec((BN, 16), lambda i: (i, 0)),
                   pl.BlockSpec((BN, 128), lambda i: (i, 0)),
                   pl.BlockSpec((BN, 16), lambda i: (i, 0))],
        out_shape=[jax.ShapeDtypeStruct((_NPAD, 128), _f32),
                   jax.ShapeDtypeStruct((_NPAD, 16), _f32),
                   jax.ShapeDtypeStruct((_NPAD, 128), _f32),
                   jax.ShapeDtypeStruct((_NPAD, 16), _f32)])(h, W)


def _node_call(h, aggs, ea0, ea1, na, Wself, WedgeP, Wattr, Wmsg_next,
               *, cat, do_silu, split_next):
    BN = 128
    d1 = Wself.shape[1]
    nagg = len(aggs)

    def body(h_ref, *rest):
        agg_refs = rest[:nagg]
        e0_ref, e1_ref, na_ref, ws_ref, we_ref, wa_ref = rest[nagg:nagg + 6]
        rest = rest[nagg + 6:]
        if Wmsg_next is not None:
            wm_ref, outs = rest[0], rest[1:]
        else:
            outs = rest
        if cat:
            agg = _cat4(*(r[...] for r in agg_refs))
        else:
            agg = agg_refs[0][...] + agg_refs[1][...]
        ea = e0_ref[...] + e1_ref[...]
        z = jnp.dot(h_ref[...], ws_ref[...], preferred_element_type=_f32)
        z = z + (agg + jnp.dot(ea, we_ref[...], preferred_element_type=_f32)) * _INV
        z = z + na_ref[...][:, :1] * wa_ref[...]
        if do_silu:
            z = _silu(z)
        outs[0][...] = z
        if Wmsg_next is not None:
            hm = jnp.dot(z, wm_ref[...], preferred_element_type=_f32)
            if split_next:
                m0, t0, m1, t1 = _split4(hm)
                outs[1][...], outs[2][...], outs[3][...], outs[4][...] = \
                    m0, t0, m1, t1
            else:
                outs[1][...] = hm

    def full(shp):
        return pl.BlockSpec(shp, lambda i: (0, 0))

    def rowblk(w):
        return pl.BlockSpec((BN, w), lambda i: (i, 0))

    in_specs = [rowblk(h.shape[1])]
    in_specs += [rowblk(a.shape[1]) for a in aggs]
    in_specs += [rowblk(16), rowblk(16), rowblk(16),
                 full(Wself.shape), full(WedgeP.shape), full(Wattr.shape)]
    args = [h, *aggs, ea0, ea1, na, Wself, WedgeP, Wattr]
    out_shape = [jax.ShapeDtypeStruct((_NPAD, d1), _f32)]
    out_specs = [rowblk(d1)]
    if Wmsg_next is not None:
        in_specs.append(full(Wmsg_next.shape))
        args.append(Wmsg_next)
        if split_next:
            out_shape += [jax.ShapeDtypeStruct((_NPAD, 128), _f32),
                          jax.ShapeDtypeStruct((_NPAD, 16), _f32),
                          jax.ShapeDtypeStruct((_NPAD, 128), _f32),
                          jax.ShapeDtypeStruct((_NPAD, 16), _f32)]
            out_specs += [rowblk(128), rowblk(16), rowblk(128), rowblk(16)]
        else:
            dn = Wmsg_next.shape[1]
            out_shape.append(jax.ShapeDtypeStruct((_NPAD, dn), _f32))
            out_specs.append(rowblk(dn))
    return pl.pallas_call(
        body, grid=(_NPAD // BN,), in_specs=in_specs, out_specs=out_specs,
        out_shape=out_shape)(*args)


def _softmax_call(pool):
    def body(p_ref, o_ref):
        p = p_ref[0] + p_ref[1]
        s = p[:_NG, :10]
        m = jnp.max(s, axis=1, keepdims=True)
        e = jnp.exp(s - m)
        o_ref[...] = e / jnp.sum(e, axis=1, keepdims=True)

    return pl.pallas_call(
        body, out_shape=jax.ShapeDtypeStruct((_NG, 10), _f32))(pool)


def _zero_buf(buf, ncols):
    z16 = jnp.zeros((16,), _f32)

    def zrow(i, cc):
        for j in range(ncols // 16):
            buf[i, pl.ds(j * 16, 16)] = z16
        return cc
    lax.fori_loop(0, buf.shape[0], zrow, 0)


def _tiled_copy(src_getter, dst_getter, nrows):
    full_copies = _RPT // nrows
    rem = _RPT - full_copies * nrows
    for t in range(full_copies):
        pltpu.sync_copy(src_getter(t * nrows, nrows), dst_getter(t * nrows, nrows))
    if rem:
        off = full_copies * nrows
        pltpu.sync_copy(src_getter(off, rem), dst_getter(off, rem))


def _edge_fs_call(hms, ws, src2, dst2):
    def body(hm0_ref, ht0_ref, hm1_ref, ht1_ref,
             wm0_ref, wt0_ref, wm1_ref, wt1_ref, src_ref, dst_ref,
             om0_ref, ot0_ref, om1_ref, ot1_ref,
             sidx, didx, rowm, rowt, wbm, wbt, accm, acct,
             semg0, semg1, semw0, semw1, sems):
        c = lax.axis_index("c")
        s = lax.axis_index("s")
        tile_row = s * _NCH
        tile_edge = s * _NCH * _CF

        _zero_buf(rowm.at[0], 128)
        _zero_buf(rowt.at[0], 16)
        _tiled_copy(lambda o, n: rowm.at[0].at[pl.ds(0, n)],
                    lambda o, n: accm.at[pl.ds(s * _RPT + o, n)], _CF)
        _tiled_copy(lambda o, n: rowt.at[0].at[pl.ds(0, n)],
                    lambda o, n: acct.at[pl.ds(s * _RPT + o, n)], _CF)
        plsc.subcore_barrier()

        semg = (semg0, semg1)
        semw = (semw0, semw1)

        def main(hm_ref, ht_ref, wm_ref, wt_ref, om_ref, ot_ref):
            def g_start(buf, ib, islot):
                pltpu.async_copy(hm_ref.at[sidx.at[ib, islot]], rowm.at[buf],
                                 semg[buf])
                pltpu.async_copy(ht_ref.at[sidx.at[ib, islot]], rowt.at[buf],
                                 semg[buf])

            def g_wait(buf):
                pltpu.make_async_copy(
                    hm_ref.at[sidx.at[0, 0]], rowm.at[buf], semg[buf]).wait()
                pltpu.make_async_copy(
                    ht_ref.at[sidx.at[0, 0]], rowt.at[buf], semg[buf]).wait()

            def w_start(ebase, buf):
                pltpu.async_copy(wm_ref.at[pl.ds(ebase, _CF)], wbm.at[buf],
                                 semw[buf])
                pltpu.async_copy(wt_ref.at[pl.ds(ebase, _CF)], wbt.at[buf],
                                 semw[buf])

            def w_wait(buf):
                pltpu.make_async_copy(
                    wm_ref.at[pl.ds(tile_edge, _CF)], wbm.at[buf], semw[buf]).wait()
                pltpu.make_async_copy(
                    wt_ref.at[pl.ds(tile_edge, _CF)], wbt.at[buf], semw[buf]).wait()

            def s_start(buf, ib, islot):
                pltpu.async_copy(rowm.at[buf], accm.at[didx.at[ib, islot]],
                                 sems, add=True)
                pltpu.async_copy(rowt.at[buf], acct.at[didx.at[ib, islot]],
                                 sems, add=True)

            def s_wait():
                pltpu.make_async_copy(
                    rowm.at[0], accm.at[didx.at[0, 0]], sems).wait()
                pltpu.make_async_copy(
                    rowt.at[0], acct.at[didx.at[0, 0]], sems).wait()

            def mul(buf):
                def mrow(i2, c2):
                    for u in range(2):
                        i = 2 * i2 + u
                        for j in range(8):
                            sl = pl.ds(j * 16, 16)
                            rowm[buf, i, sl] = rowm[buf, i, sl] * wbm[buf, i, sl]
                        sl = pl.ds(0, 16)
                        rowt[buf, i, sl] = rowt[buf, i, sl] * wbt[buf, i, sl]
                    return c2
                lax.fori_loop(0, _CF // 2, mrow, 0)

            pltpu.sync_copy(src_ref.at[pl.ds(tile_row, 8)], sidx.at[0])
            pltpu.sync_copy(dst_ref.at[pl.ds(tile_row, 8)], didx.at[0])
            g_start(0, 0, 0)
            w_start(tile_edge, 0)

            def pair(p, cc):
                a = 2 * p
                b = a + 1
                q = lax.rem(p, 4)
                grp = lax.div(p, 4)

                @pl.when(jnp.logical_and(q == 0, grp + 1 < _NGRP))
                def _():
                    row = tile_row + (grp + 1) * 8
                    pltpu.sync_copy(src_ref.at[pl.ds(row, 8)],
                                    sidx.at[lax.rem(grp + 1, 2)])
                    pltpu.sync_copy(dst_ref.at[pl.ds(row, 8)],
                                    didx.at[lax.rem(grp + 1, 2)])

                ib = lax.rem(grp, 2)

                @pl.when(p > 0)
                def _():
                    s_wait()
                g_start(1, ib, lax.rem(b, 8))
                w_start(tile_edge + b * _CF, 1)
                g_wait(0)
                w_wait(0)
                mul(0)
                s_start(0, ib, lax.rem(a, 8))

                s_wait()
                @pl.when(p + 1 < _NPAIR)
                def _():
                    nk = a + 2
                    g_start(0, lax.rem(lax.div(nk, 8), 2), lax.rem(nk, 8))
                    w_start(tile_edge + nk * _CF, 0)
                g_wait(1)
                w_wait(1)
                mul(1)
                s_start(1, ib, lax.rem(b, 8))
                return cc

            lax.fori_loop(0, _NPAIR, pair, 0)
            s_wait()
            plsc.subcore_barrier()
            _tiled_copy(lambda o, n: accm.at[pl.ds(s * _RPT + o, n)],
                        lambda o, n: om_ref.at[pl.ds(s * _RPT + o, n)], 128)
            _tiled_copy(lambda o, n: acct.at[pl.ds(s * _RPT + o, n)],
                        lambda o, n: ot_ref.at[pl.ds(s * _RPT + o, n)], 128)

        @pl.when(c == 0)
        def _():
            main(hm0_ref, ht0_ref, wm0_ref, wt0_ref, om0_ref, ot0_ref)

        @pl.when(c == 1)
        def _():
            main(hm1_ref, ht1_ref, wm1_ref, wt1_ref, om1_ref, ot1_ref)

    return pl.kernel(
        body,
        out_type=[jax.ShapeDtypeStruct((_NPAD, 128), _f32),
                  jax.ShapeDtypeStruct((_NPAD, 16), _f32),
                  jax.ShapeDtypeStruct((_NPAD, 128), _f32),
                  jax.ShapeDtypeStruct((_NPAD, 16), _f32)],
        mesh=_MESH,
        compiler_params=_SC_PARAMS,
        scratch_types=[
            pltpu.VMEM((2, 8, _CF), jnp.int32),
            pltpu.VMEM((2, 8, _CF), jnp.int32),
            pltpu.VMEM((2, _CF, 128), _f32),
            pltpu.VMEM((2, _CF, 16), _f32),
            pltpu.VMEM((2, _CF, 128), _f32),
            pltpu.VMEM((2, _CF, 16), _f32),
            pltpu.VMEM_SHARED((_NPAD, 128), _f32),
            pltpu.VMEM_SHARED((_NPAD, 16), _f32),
            pltpu.SemaphoreType.DMA,
            pltpu.SemaphoreType.DMA,
            pltpu.SemaphoreType.DMA,
            pltpu.SemaphoreType.DMA,
            pltpu.SemaphoreType.DMA,
        ])(*hms, *ws, src2, dst2)


def _edge_es_call(hm4, w4, srcp, dstp):
    NCH = _EPAD // (32 * _C)

    def body(hm_ref, w_ref, src_ref, dst_ref, out_ref,
             sidx, didx, rows, wbuf, acc, sem):
        c = lax.axis_index("c")
        s = lax.axis_index("s")
        wid = c * 16 + s

        _zero_buf(rows, 16)
        _tiled_copy(lambda o, n: rows.at[pl.ds(0, n)],
                    lambda o, n: acc.at[pl.ds(s * _RPT + o, n)], _C)
        plsc.subcore_barrier()

        def chunk(k, cc):
            base = (wid * NCH + k) * _C
            pltpu.sync_copy(src_ref.at[pl.ds(base, _C)], sidx)
            pltpu.sync_copy(dst_ref.at[pl.ds(base, _C)], didx)
            cp = pltpu.async_copy(hm_ref.at[sidx], rows, sem)
            pltpu.sync_copy(w_ref.at[pl.ds(base, _C)], wbuf)
            cp.wait()

            def mrow(i, c2):
                sl = pl.ds(0, 16)
                rows[i, sl] = rows[i, sl] * wbuf[i, sl]
                return c2
            lax.fori_loop(0, _C, mrow, 0)
            pltpu.sync_copy(rows, acc.at[didx], add=True)
            return cc
        lax.fori_loop(0, NCH, chunk, 0)
        plsc.subcore_barrier()
        _tiled_copy(lambda o, n: acc.at[pl.ds(s * _RPT + o, n)],
                    lambda o, n: out_ref.at[c, pl.ds(s * _RPT + o, n)], 128)

    return pl.kernel(
        body,
        out_type=jax.ShapeDtypeStruct((2, _NPAD, 16), _f32),
        mesh=_MESH,
        compiler_params=_SC_PARAMS,
        scratch_types=[
            pltpu.VMEM((_C,), jnp.int32),
            pltpu.VMEM((_C,), jnp.int32),
            pltpu.VMEM((_C, 16), _f32),
            pltpu.VMEM((_C, 16), _f32),
            pltpu.VMEM_SHARED((_NPAD, 16), _f32),
            pltpu.SemaphoreType.DMA,
        ])(hm4, w4, srcp, dstp)


def _ea_call(eap, dstp):
    NCH = _EPAD // (32 * _C)

    def body(ea_ref, dst_ref, out_ref, didx, rows, acc, sem):
        c = lax.axis_index("c")
        s = lax.axis_index("s")
        wid = c * 16 + s

        _zero_buf(rows, 16)
        _tiled_copy(lambda o, n: rows.at[pl.ds(0, n)],
                    lambda o, n: acc.at[pl.ds(s * _RPT + o, n)], _C)
        plsc.subcore_barrier()

        def chunk(k, cc):
            base = (wid * NCH + k) * _C
            pltpu.sync_copy(dst_ref.at[pl.ds(base, _C)], didx)
            pltpu.sync_copy(ea_ref.at[pl.ds(base, _C)], rows)
            pltpu.sync_copy(rows, acc.at[didx], add=True)
            return cc
        lax.fori_loop(0, NCH, chunk, 0)
        plsc.subcore_barrier()
        _tiled_copy(lambda o, n: acc.at[pl.ds(s * _RPT + o, n)],
                    lambda o, n: out_ref.at[c, pl.ds(s * _RPT + o, n)], 128)

    return pl.kernel(
        body,
        out_type=jax.ShapeDtypeStruct((2, _NPAD, 16), _f32),
        mesh=_MESH,
        compiler_params=_SC_PARAMS,
        scratch_types=[
            pltpu.VMEM((_C,), jnp.int32),
            pltpu.VMEM((_C, 16), _f32),
            pltpu.VMEM_SHARED((_NPAD, 16), _f32),
            pltpu.SemaphoreType.DMA,
        ])(eap, dstp)


def _pool_call(h4, batchp):
    CP = 64
    NCHT = _NPAD // CP

    def body(h_ref, b_ref, out_ref, bidx, rows, zbuf, acc, sem):
        c = lax.axis_index("c")
        s = lax.axis_index("s")
        wid = c * 16 + s

        @pl.when(s == 0)
        def _():
            _zero_buf(zbuf, 16)
            pltpu.sync_copy(zbuf, acc)
        plsc.subcore_barrier()

        def chunk(k, cc):
            idx = k * 32 + wid

            @pl.when(idx < NCHT)
            def _():
                base = idx * CP
                pltpu.sync_copy(b_ref.at[pl.ds(base, CP)], bidx)
                pltpu.sync_copy(h_ref.at[pl.ds(base, CP)], rows)
                pltpu.sync_copy(rows, acc.at[bidx], add=True)
            return cc
        lax.fori_loop(0, (NCHT + 31) // 32, chunk, 0)
        plsc.subcore_barrier()

        @pl.when(s == 0)
        def _():
            pltpu.sync_copy(acc, out_ref.at[c])

    return pl.kernel(
        body,
        out_type=jax.ShapeDtypeStruct((2, 72, 16), _f32),
        mesh=_MESH,
        compiler_params=_SC_PARAMS,
        scratch_types=[
            pltpu.VMEM((CP,), jnp.int32),
            pltpu.VMEM((CP, 16), _f32),
            pltpu.VMEM((72, 16), _f32),
            pltpu.VMEM_SHARED((72, 16), _f32),
            pltpu.SemaphoreType.DMA,
        ])(h4, batchp)


def kernel(x, node_attr, edge_src, edge_dst, edge_attr, edge_length_embedding,
           batch, params):
    xp = jnp.zeros((_NPAD, 128), _f32).at[:_N].set(x)
    nap = jnp.zeros((_NPAD, 16), _f32).at[:_N].set(
        jnp.broadcast_to(node_attr, (_N, 16)))
    srcp = jnp.full((_EPAD,), _N, jnp.int32).at[:_E].set(edge_src.astype(jnp.int32))
    dstp = jnp.full((_EPAD,), _N, jnp.int32).at[:_E].set(edge_dst.astype(jnp.int32))
    src2 = srcp.reshape(_EPAD // _CF, _CF)
    dst2 = dstp.reshape(_EPAD // _CF, _CF)
    eap = jnp.zeros((_EPAD, 16), _f32).at[:_E, :9].set(edge_attr)
    embT = jnp.zeros((10, _EPAD), _f32).at[:, :_E].set(edge_length_embedding.T)
    batchp = jnp.full((_NPAD,), _NG, jnp.int32).at[:_N].set(batch.astype(jnp.int32))

    def wedgeP(p):
        return jnp.zeros((16, p['W_edge'].shape[1]), _f32).at[:9].set(p['W_edge'])

    p4 = params[3]
    Wself4 = jnp.zeros((288, 16), _f32).at[:, :10].set(p4['W_self'])
    Wedge4 = jnp.zeros((16, 16), _f32).at[:9, :10].set(p4['W_edge'])
    Wattr4 = jnp.zeros((1, 16), _f32).at[:, :10].set(p4['W_attr'])
    fcW3_4 = jnp.zeros((128, 16), _f32).at[:, :10].set(p4['fc_W3'])
    Wmsg4 = jnp.zeros((288, 16), _f32).at[:, :10].set(p4['W_msg'])

    ea_pair = _ea_call(eap, dstp)
    ea0, ea1 = ea_pair[0], ea_pair[1]
    all_ws = [_radial_call(embT, params[li]['fc_W0'], params[li]['fc_W1'],
                           params[li]['fc_W2'], params[li]['fc_W3'], split=True)
              for li in range(3)]
    (w4,) = _radial_call(embT, p4['fc_W0'], p4['fc_W1'], p4['fc_W2'], fcW3_4,
                         split=False)

    hms = _msg_split_call(xp, params[0]['W_msg'])
    h = xp
    for li in range(3):
        p = params[li]
        ws = all_ws[li]
        aggs = _edge_fs_call(hms, ws, src2, dst2)
        if li < 2:
            h, *hms = _node_call(
                h, aggs, ea0, ea1, nap, p['W_self'], wedgeP(p),
                p['W_attr'], params[li + 1]['W_msg'],
                cat=True, do_silu=True, split_next=True)
        else:
            h, hm4 = _node_call(
                h, aggs, ea0, ea1, nap, p['W_self'], wedgeP(p),
                p['W_attr'], Wmsg4, cat=True, do_silu=True,
                split_next=False)

    agg4 = _edge_es_call(hm4, w4, srcp, dstp)
    (h4,) = _node_call(h, [agg4[0], agg4[1]], ea0, ea1, nap, Wself4, Wedge4,
                       Wattr4, None, cat=False, do_silu=False, split_next=False)
    pool = _pool_call(h4, batchp)
    return _softmax_call(pool)


# speedup vs baseline: 1.1206x; 1.0130x over previous
import numpy as np
import jax
import jax.numpy as jnp
from jax import lax
from jax.experimental import pallas as pl
from jax.experimental.pallas import tpu as pltpu
from jax.experimental.pallas import tpu_sc as plsc

_N = 10000
_NG = 64
_NPAD = 10112
_E = 320000
_EPAD = 327680
_C = 128
_CF = 64
_RPT = _NPAD // 16
_NCH = _EPAD // (16 * _CF)
_NPAIR = _NCH // 2
_NGRP = _NCH // 8
_INV = 1.0 / np.sqrt(32.0)

_MESH = plsc.VectorSubcoreMesh(
    core_axis_name="c", subcore_axis_name="s", num_cores=2, num_subcores=16)
_SC_PARAMS = pltpu.CompilerParams(use_tc_tiling_on_sc=False,
                                  needs_layout_passes=False)

_f32 = jnp.float32


def _silu(v):
    return v * lax.logistic(v)


def _split4(v):
    return v[:, :128], v[:, 128:144], v[:, 160:288], v[:, 144:160]


def _cat4(m0, t0, m1, t1):
    return jnp.concatenate([m0, t0, t1, m1], axis=1)


def _radial_call(embT, W0, W1, W2, W3, split):
    BE = 1024
    d1 = W3.shape[1]

    def body(emb_ref, w0_ref, w1_ref, w2_ref, w3_ref, *outs):
        v = _silu(lax.dot_general(
            emb_ref[...], w0_ref[...], (((0,), (0,)), ((), ())),
            preferred_element_type=_f32))
        v = _silu(jnp.dot(v, w1_ref[...], preferred_element_type=_f32))
        v = _silu(jnp.dot(v, w2_ref[...], preferred_element_type=_f32))
        v = jnp.dot(v, w3_ref[...], preferred_element_type=_f32)
        if split:
            m0, t0, m1, t1 = _split4(v)
            outs[0][...], outs[1][...], outs[2][...], outs[3][...] = m0, t0, m1, t1
        else:
            outs[0][...] = v

    def full(shp):
        return pl.BlockSpec(shp, lambda i: (0, 0))

    in_specs = [pl.BlockSpec((embT.shape[0], BE), lambda i: (0, i)),
                full(W0.shape), full(W1.shape), full(W2.shape), full(W3.shape)]
    if split:
        out_shape = [jax.ShapeDtypeStruct((_EPAD, 128), _f32),
                     jax.ShapeDtypeStruct((_EPAD, 16), _f32),
                     jax.ShapeDtypeStruct((_EPAD, 128), _f32),
                     jax.ShapeDtypeStruct((_EPAD, 16), _f32)]
        out_specs = [pl.BlockSpec((BE, 128), lambda i: (i, 0)),
                     pl.BlockSpec((BE, 16), lambda i: (i, 0)),
                     pl.BlockSpec((BE, 128), lambda i: (i, 0)),
                     pl.BlockSpec((BE, 16), lambda i: (i, 0))]
    else:
        out_shape = [jax.ShapeDtypeStruct((_EPAD, d1), _f32)]
        out_specs = [pl.BlockSpec((BE, d1), lambda i: (i, 0))]
    return pl.pallas_call(
        body, grid=(_EPAD // BE,), in_specs=in_specs, out_specs=out_specs,
        out_shape=out_shape)(embT, W0, W1, W2, W3)


def _msg_split_call(h, W):
    BN = 128

    def body(h_ref, w_ref, o0, o1, o2, o3):
        hm = jnp.dot(h_ref[...], w_ref[...], preferred_element_type=_f32)
        m0, t0, m1, t1 = _split4(hm)
        o0[...], o1[...], o2[...], o3[...] = m0, t0, m1, t1

    return pl.pallas_call(
        body, grid=(_NPAD // BN,),
        in_specs=[pl.BlockSpec((BN, h.shape[1]), lambda i: (i, 0)),
                  pl.BlockSpec(W.shape, lambda i: (0, 0))],
        out_specs=[pl.BlockSpec((BN, 128), lambda i: (i, 0)),
                   pl.BlockSpec((BN, 16), lambda i: (i, 0)),
                   pl.BlockSpec((BN, 128), lambda i: (i, 0)),
                   pl.BlockSpec((BN, 16), lambda i: (i, 0))],
        out_shape=[jax.ShapeDtypeStruct((_NPAD, 128), _f32),
                   jax.ShapeDtypeStruct((_NPAD, 16), _f32),
                   jax.ShapeDtypeStruct((_NPAD, 128), _f32),
                   jax.ShapeDtypeStruct((_NPAD, 16), _f32)])(h, W)


def _node_call(h, aggs, ea0, ea1, na, Wself, WedgeP, Wattr, Wmsg_next,
               *, cat, do_silu, split_next):
    BN = 128
    d1 = Wself.shape[1]
    nagg = len(aggs)

    def body(h_ref, *rest):
        agg_refs = rest[:nagg]
        e0_ref, e1_ref, na_ref, ws_ref, we_ref, wa_ref = rest[nagg:nagg + 6]
        rest = rest[nagg + 6:]
        if Wmsg_next is not None:
            wm_ref, outs = rest[0], rest[1:]
        else:
            outs = rest
        if cat:
            agg = _cat4(*(r[...] for r in agg_refs))
        else:
            agg = agg_refs[0][...] + agg_refs[1][...]
        ea = e0_ref[...] + e1_ref[...]
        z = jnp.dot(h_ref[...], ws_ref[...], preferred_element_type=_f32)
        z = z + (agg + jnp.dot(ea, we_ref[...], preferred_element_type=_f32)) * _INV
        z = z + na_ref[...][:, :1] * wa_ref[...]
        if do_silu:
            z = _silu(z)
        outs[0][...] = z
        if Wmsg_next is not None:
            hm = jnp.dot(z, wm_ref[...], preferred_element_type=_f32)
            if split_next:
                m0, t0, m1, t1 = _split4(hm)
                outs[1][...], outs[2][...], outs[3][...], outs[4][...] = \
                    m0, t0, m1, t1
            else:
                outs[1][...] = hm

    def full(shp):
        return pl.BlockSpec(shp, lambda i: (0, 0))

    def rowblk(w):
        return pl.BlockSpec((BN, w), lambda i: (i, 0))

    in_specs = [rowblk(h.shape[1])]
    in_specs += [rowblk(a.shape[1]) for a in aggs]
    in_specs += [rowblk(16), rowblk(16), rowblk(16),
                 full(Wself.shape), full(WedgeP.shape), full(Wattr.shape)]
    args = [h, *aggs, ea0, ea1, na, Wself, WedgeP, Wattr]
    out_shape = [jax.ShapeDtypeStruct((_NPAD, d1), _f32)]
    out_specs = [rowblk(d1)]
    if Wmsg_next is not None:
        in_specs.append(full(Wmsg_next.shape))
        args.append(Wmsg_next)
        if split_next:
            out_shape += [jax.ShapeDtypeStruct((_NPAD, 128), _f32),
                          jax.ShapeDtypeStruct((_NPAD, 16), _f32),
                          jax.ShapeDtypeStruct((_NPAD, 128), _f32),
                          jax.ShapeDtypeStruct((_NPAD, 16), _f32)]
            out_specs += [rowblk(128), rowblk(16), rowblk(128), rowblk(16)]
        else:
            dn = Wmsg_next.shape[1]
            out_shape.append(jax.ShapeDtypeStruct((_NPAD, dn), _f32))
            out_specs.append(rowblk(dn))
    return pl.pallas_call(
        body, grid=(_NPAD // BN,), in_specs=in_specs, out_specs=out_specs,
        out_shape=out_shape)(*args)


def _softmax_call(pool):
    def body(p_ref, o_ref):
        p = p_ref[0] + p_ref[1]
        s = p[:_NG, :10]
        m = jnp.max(s, axis=1, keepdims=True)
        e = jnp.exp(s - m)
        o_ref[...] = e / jnp.sum(e, axis=1, keepdims=True)

    return pl.pallas_call(
        body, out_shape=jax.ShapeDtypeStruct((_NG, 10), _f32))(pool)


def _zero_buf(buf, ncols):
    z16 = jnp.zeros((16,), _f32)

    def zrow(i, cc):
        for j in range(ncols // 16):
            buf[i, pl.ds(j * 16, 16)] = z16
        return cc
    lax.fori_loop(0, buf.shape[0], zrow, 0)


def _tiled_copy(src_getter, dst_getter, nrows):
    full_copies = _RPT // nrows
    rem = _RPT - full_copies * nrows
    for t in range(full_copies):
        pltpu.sync_copy(src_getter(t * nrows, nrows), dst_getter(t * nrows, nrows))
    if rem:
        off = full_copies * nrows
        pltpu.sync_copy(src_getter(off, rem), dst_getter(off, rem))


def _edge_fs_call(hms, ws, src2, dst2):
    def body(hm0_ref, ht0_ref, hm1_ref, ht1_ref,
             wm0_ref, wt0_ref, wm1_ref, wt1_ref, src_ref, dst_ref,
             om0_ref, ot0_ref, om1_ref, ot1_ref,
             sidx, didx, rowm, rowt, wbm, wbt, accm, acct,
             semg0, semg1, semw0, semw1, sems, semi):
        c = lax.axis_index("c")
        s = lax.axis_index("s")
        tile_row = s * _NCH
        tile_edge = s * _NCH * _CF

        _zero_buf(rowm.at[0], 128)
        _zero_buf(rowt.at[0], 16)
        _tiled_copy(lambda o, n: rowm.at[0].at[pl.ds(0, n)],
                    lambda o, n: accm.at[pl.ds(s * _RPT + o, n)], _CF)
        _tiled_copy(lambda o, n: rowt.at[0].at[pl.ds(0, n)],
                    lambda o, n: acct.at[pl.ds(s * _RPT + o, n)], _CF)
        plsc.subcore_barrier()

        semg = (semg0, semg1)
        semw = (semw0, semw1)

        def main(hm_ref, ht_ref, wm_ref, wt_ref, om_ref, ot_ref):
            def g_start(buf, ib, islot):
                pltpu.async_copy(hm_ref.at[sidx.at[ib, islot]], rowm.at[buf],
                                 semg[buf])
                pltpu.async_copy(ht_ref.at[sidx.at[ib, islot]], rowt.at[buf],
                                 semg[buf])

            def g_wait(buf):
                pltpu.make_async_copy(
                    hm_ref.at[sidx.at[0, 0]], rowm.at[buf], semg[buf]).wait()
                pltpu.make_async_copy(
                    ht_ref.at[sidx.at[0, 0]], rowt.at[buf], semg[buf]).wait()

            def w_start(ebase, buf):
                pltpu.async_copy(wm_ref.at[pl.ds(ebase, _CF)], wbm.at[buf],
                                 semw[buf])
                pltpu.async_copy(wt_ref.at[pl.ds(ebase, _CF)], wbt.at[buf],
                                 semw[buf])

            def w_wait(buf):
                pltpu.make_async_copy(
                    wm_ref.at[pl.ds(tile_edge, _CF)], wbm.at[buf], semw[buf]).wait()
                pltpu.make_async_copy(
                    wt_ref.at[pl.ds(tile_edge, _CF)], wbt.at[buf], semw[buf]).wait()

            def s_start(buf, ib, islot):
                pltpu.async_copy(rowm.at[buf], accm.at[didx.at[ib, islot]],
                                 sems, add=True)
                pltpu.async_copy(rowt.at[buf], acct.at[didx.at[ib, islot]],
                                 sems, add=True)

            def s_wait():
                pltpu.make_async_copy(
                    rowm.at[0], accm.at[didx.at[0, 0]], sems).wait()
                pltpu.make_async_copy(
                    rowt.at[0], acct.at[didx.at[0, 0]], sems).wait()

            def mul(buf):
                def mrow(i2, c2):
                    for u in range(2):
                        i = 2 * i2 + u
                        for j in range(8):
                            sl = pl.ds(j * 16, 16)
                            rowm[buf, i, sl] = rowm[buf, i, sl] * wbm[buf, i, sl]
                        sl = pl.ds(0, 16)
                        rowt[buf, i, sl] = rowt[buf, i, sl] * wbt[buf, i, sl]
                    return c2
                lax.fori_loop(0, _CF // 2, mrow, 0)

            pltpu.sync_copy(src_ref.at[pl.ds(tile_row, 8)], sidx.at[0])
            pltpu.sync_copy(dst_ref.at[pl.ds(tile_row, 8)], didx.at[0])
            g_start(0, 0, 0)
            w_start(tile_edge, 0)

            def pair(p, cc):
                a = 2 * p
                b = a + 1
                q = lax.rem(p, 4)
                grp = lax.div(p, 4)

                @pl.when(jnp.logical_and(q == 0, grp + 1 < _NGRP))
                def _():
                    row = tile_row + (grp + 1) * 8
                    pltpu.async_copy(src_ref.at[pl.ds(row, 8)],
                                     sidx.at[lax.rem(grp + 1, 2)], semi)
                    pltpu.async_copy(dst_ref.at[pl.ds(row, 8)],
                                     didx.at[lax.rem(grp + 1, 2)], semi)

                @pl.when(jnp.logical_and(q == 3, grp + 1 < _NGRP))
                def _():
                    pltpu.make_async_copy(src_ref.at[pl.ds(tile_row, 8)],
                                          sidx.at[0], semi).wait()
                    pltpu.make_async_copy(dst_ref.at[pl.ds(tile_row, 8)],
                                          didx.at[0], semi).wait()

                ib = lax.rem(grp, 2)

                @pl.when(p > 0)
                def _():
                    s_wait()
                g_start(1, ib, lax.rem(b, 8))
                w_start(tile_edge + b * _CF, 1)
                g_wait(0)
                w_wait(0)
                mul(0)
                s_start(0, ib, lax.rem(a, 8))

                s_wait()
                @pl.when(p + 1 < _NPAIR)
                def _():
                    nk = a + 2
                    g_start(0, lax.rem(lax.div(nk, 8), 2), lax.rem(nk, 8))
                    w_start(tile_edge + nk * _CF, 0)
                g_wait(1)
                w_wait(1)
                mul(1)
                s_start(1, ib, lax.rem(b, 8))
                return cc

            lax.fori_loop(0, _NPAIR, pair, 0)
            s_wait()
            plsc.subcore_barrier()
            _tiled_copy(lambda o, n: accm.at[pl.ds(s * _RPT + o, n)],
                        lambda o, n: om_ref.at[pl.ds(s * _RPT + o, n)], 128)
            _tiled_copy(lambda o, n: acct.at[pl.ds(s * _RPT + o, n)],
                        lambda o, n: ot_ref.at[pl.ds(s * _RPT + o, n)], 128)

        @pl.when(c == 0)
        def _():
            main(hm0_ref, ht0_ref, wm0_ref, wt0_ref, om0_ref, ot0_ref)

        @pl.when(c == 1)
        def _():
            main(hm1_ref, ht1_ref, wm1_ref, wt1_ref, om1_ref, ot1_ref)

    return pl.kernel(
        body,
        out_type=[jax.ShapeDtypeStruct((_NPAD, 128), _f32),
                  jax.ShapeDtypeStruct((_NPAD, 16), _f32),
                  jax.ShapeDtypeStruct((_NPAD, 128), _f32),
                  jax.ShapeDtypeStruct((_NPAD, 16), _f32)],
        mesh=_MESH,
        compiler_params=_SC_PARAMS,
        scratch_types=[
            pltpu.VMEM((2, 8, _CF), jnp.int32),
            pltpu.VMEM((2, 8, _CF), jnp.int32),
            pltpu.VMEM((2, _CF, 128), _f32),
            pltpu.VMEM((2, _CF, 16), _f32),
            pltpu.VMEM((2, _CF, 128), _f32),
            pltpu.VMEM((2, _CF, 16), _f32),
            pltpu.VMEM_SHARED((_NPAD, 128), _f32),
            pltpu.VMEM_SHARED((_NPAD, 16), _f32),
            pltpu.SemaphoreType.DMA,
            pltpu.SemaphoreType.DMA,
            pltpu.SemaphoreType.DMA,
            pltpu.SemaphoreType.DMA,
            pltpu.SemaphoreType.DMA,
            pltpu.SemaphoreType.DMA,
        ])(*hms, *ws, src2, dst2)


def _edge_es_call(hm4, w4, srcp, dstp):
    NCH = _EPAD // (32 * _C)

    def body(hm_ref, w_ref, src_ref, dst_ref, out_ref,
             sidx, didx, rows, wbuf, acc, sem):
        c = lax.axis_index("c")
        s = lax.axis_index("s")
        wid = c * 16 + s

        _zero_buf(rows, 16)
        _tiled_copy(lambda o, n: rows.at[pl.ds(0, n)],
                    lambda o, n: acc.at[pl.ds(s * _RPT + o, n)], _C)
        plsc.subcore_barrier()

        def chunk(k, cc):
            base = (wid * NCH + k) * _C
            pltpu.sync_copy(src_ref.at[pl.ds(base, _C)], sidx)
            pltpu.sync_copy(dst_ref.at[pl.ds(base, _C)], didx)
            cp = pltpu.async_copy(hm_ref.at[sidx], rows, sem)
            pltpu.sync_copy(w_ref.at[pl.ds(base, _C)], wbuf)
            cp.wait()

            def mrow(i, c2):
                sl = pl.ds(0, 16)
                rows[i, sl] = rows[i, sl] * wbuf[i, sl]
                return c2
            lax.fori_loop(0, _C, mrow, 0)
            pltpu.sync_copy(rows, acc.at[didx], add=True)
            return cc
        lax.fori_loop(0, NCH, chunk, 0)
        plsc.subcore_barrier()
        _tiled_copy(lambda o, n: acc.at[pl.ds(s * _RPT + o, n)],
                    lambda o, n: out_ref.at[c, pl.ds(s * _RPT + o, n)], 128)

    return pl.kernel(
        body,
        out_type=jax.ShapeDtypeStruct((2, _NPAD, 16), _f32),
        mesh=_MESH,
        compiler_params=_SC_PARAMS,
        scratch_types=[
            pltpu.VMEM((_C,), jnp.int32),
            pltpu.VMEM((_C,), jnp.int32),
            pltpu.VMEM((_C, 16), _f32),
            pltpu.VMEM((_C, 16), _f32),
            pltpu.VMEM_SHARED((_NPAD, 16), _f32),
            pltpu.SemaphoreType.DMA,
        ])(hm4, w4, srcp, dstp)


def _ea_call(eap, dstp):
    NCH = _EPAD // (32 * _C)

    def body(ea_ref, dst_ref, out_ref, didx, rows, acc, sem):
        c = lax.axis_index("c")
        s = lax.axis_index("s")
        wid = c * 16 + s

        _zero_buf(rows, 16)
        _tiled_copy(lambda o, n: rows.at[pl.ds(0, n)],
                    lambda o, n: acc.at[pl.ds(s * _RPT + o, n)], _C)
        plsc.subcore_barrier()

        def chunk(k, cc):
            base = (wid * NCH + k) * _C
            pltpu.sync_copy(dst_ref.at[pl.ds(base, _C)], didx)
            pltpu.sync_copy(ea_ref.at[pl.ds(base, _C)], rows)
            pltpu.sync_copy(rows, acc.at[didx], add=True)
            return cc
        lax.fori_loop(0, NCH, chunk, 0)
        plsc.subcore_barrier()
        _tiled_copy(lambda o, n: acc.at[pl.ds(s * _RPT + o, n)],
                    lambda o, n: out_ref.at[c, pl.ds(s * _RPT + o, n)], 128)

    return pl.kernel(
        body,
        out_type=jax.ShapeDtypeStruct((2, _NPAD, 16), _f32),
        mesh=_MESH,
        compiler_params=_SC_PARAMS,
        scratch_types=[
            pltpu.VMEM((_C,), jnp.int32),
            pltpu.VMEM((_C, 16), _f32),
            pltpu.VMEM_SHARED((_NPAD, 16), _f32),
            pltpu.SemaphoreType.DMA,
        ])(eap, dstp)


def _pool_call(h4, batchp):
    CP = 64
    NCHT = _NPAD // CP

    def body(h_ref, b_ref, out_ref, bidx, rows, zbuf, acc, sem):
        c = lax.axis_index("c")
        s = lax.axis_index("s")
        wid = c * 16 + s

        @pl.when(s == 0)
        def _():
            _zero_buf(zbuf, 16)
            pltpu.sync_copy(zbuf, acc)
        plsc.subcore_barrier()

        def chunk(k, cc):
            idx = k * 32 + wid

            @pl.when(idx < NCHT)
            def _():
                base = idx * CP
                pltpu.sync_copy(b_ref.at[pl.ds(base, CP)], bidx)
                pltpu.sync_copy(h_ref.at[pl.ds(base, CP)], rows)
                pltpu.sync_copy(rows, acc.at[bidx], add=True)
            return cc
        lax.fori_loop(0, (NCHT + 31) // 32, chunk, 0)
        plsc.subcore_barrier()

        @pl.when(s == 0)
        def _():
            pltpu.sync_copy(acc, out_ref.at[c])

    return pl.kernel(
        body,
        out_type=jax.ShapeDtypeStruct((2, 72, 16), _f32),
        mesh=_MESH,
        compiler_params=_SC_PARAMS,
        scratch_types=[
            pltpu.VMEM((CP,), jnp.int32),
            pltpu.VMEM((CP, 16), _f32),
            pltpu.VMEM((72, 16), _f32),
            pltpu.VMEM_SHARED((72, 16), _f32),
            pltpu.SemaphoreType.DMA,
        ])(h4, batchp)


def kernel(x, node_attr, edge_src, edge_dst, edge_attr, edge_length_embedding,
           batch, params):
    xp = jnp.zeros((_NPAD, 128), _f32).at[:_N].set(x)
    nap = jnp.zeros((_NPAD, 16), _f32).at[:_N].set(
        jnp.broadcast_to(node_attr, (_N, 16)))
    srcp = jnp.full((_EPAD,), _N, jnp.int32).at[:_E].set(edge_src.astype(jnp.int32))
    dstp = jnp.full((_EPAD,), _N, jnp.int32).at[:_E].set(edge_dst.astype(jnp.int32))
    src2 = srcp.reshape(_EPAD // _CF, _CF)
    dst2 = dstp.reshape(_EPAD // _CF, _CF)
    eap = jnp.zeros((_EPAD, 16), _f32).at[:_E, :9].set(edge_attr)
    embT = jnp.zeros((10, _EPAD), _f32).at[:, :_E].set(edge_length_embedding.T)
    batchp = jnp.full((_NPAD,), _NG, jnp.int32).at[:_N].set(batch.astype(jnp.int32))

    def wedgeP(p):
        return jnp.zeros((16, p['W_edge'].shape[1]), _f32).at[:9].set(p['W_edge'])

    p4 = params[3]
    Wself4 = jnp.zeros((288, 16), _f32).at[:, :10].set(p4['W_self'])
    Wedge4 = jnp.zeros((16, 16), _f32).at[:9, :10].set(p4['W_edge'])
    Wattr4 = jnp.zeros((1, 16), _f32).at[:, :10].set(p4['W_attr'])
    fcW3_4 = jnp.zeros((128, 16), _f32).at[:, :10].set(p4['fc_W3'])
    Wmsg4 = jnp.zeros((288, 16), _f32).at[:, :10].set(p4['W_msg'])

    ea_pair = _ea_call(eap, dstp)
    ea0, ea1 = ea_pair[0], ea_pair[1]
    all_ws = [_radial_call(embT, params[li]['fc_W0'], params[li]['fc_W1'],
                           params[li]['fc_W2'], params[li]['fc_W3'], split=True)
              for li in range(3)]
    (w4,) = _radial_call(embT, p4['fc_W0'], p4['fc_W1'], p4['fc_W2'], fcW3_4,
                         split=False)

    hms = _msg_split_call(xp, params[0]['W_msg'])
    h = xp
    for li in range(3):
        p = params[li]
        ws = all_ws[li]
        aggs = _edge_fs_call(hms, ws, src2, dst2)
        if li < 2:
            h, *hms = _node_call(
                h, aggs, ea0, ea1, nap, p['W_self'], wedgeP(p),
                p['W_attr'], params[li + 1]['W_msg'],
                cat=True, do_silu=True, split_next=True)
        else:
            h, hm4 = _node_call(
                h, aggs, ea0, ea1, nap, p['W_self'], wedgeP(p),
                p['W_attr'], Wmsg4, cat=True, do_silu=True,
                split_next=False)

    agg4 = _edge_es_call(hm4, w4, srcp, dstp)
    (h4,) = _node_call(h, [agg4[0], agg4[1]], ea0, ea1, nap, Wself4, Wedge4,
                       Wattr4, None, cat=False, do_silu=False, split_next=False)
    pool = _pool_call(h4, batchp)
    return _softmax_call(pool)
